# Initial kernel scaffold; baseline (speedup 1.0000x reference)
import jax
import jax.numpy as jnp
from jax.experimental import pallas as pl


def kernel(pos, h, eps, t, conditions, W_in, Wc, We1, We2, Wx, Wh, Wout, combined_mask, edge_index):
    raise NotImplementedError("write your pallas kernel here")


# speedup vs baseline: 2.8763x; 2.8763x over previous
import dataclasses
import functools

import jax
import jax.numpy as jnp
from jax import lax
from jax.experimental import pallas as pl
from jax.experimental.pallas import tpu as pltpu
from jax.experimental.pallas import tpu_sc as plsc

N_NODES = 10000
N_EDGES = 160000
NSEG = 256
HID = 256
TSTEPS = 1000.0

NB = 2000
EB = 2000
GB = 128
SB = 80
TW = 384


def _silu(x):
    return x * lax.logistic(x)


def _onehot(m, rows):
    return (m == lax.broadcasted_iota(jnp.int32, (rows, NSEG), 1)).astype(
        jnp.float32)


def _t1a_body(mask_ref, eps_ref, t_ref, mean_ref, sc_ref):
    o = _onehot(mask_ref[...], N_NODES)
    ones = jnp.ones((N_NODES, 1), jnp.float32)
    cdims = (((0,), (0,)), ((), ()))
    counts = jnp.maximum(lax.dot_general(o, ones, cdims), 1.0)
    sums = lax.dot_general(o, eps_ref[...], cdims)
    mean_ref[...] = sums / counts
    t = t_ref[...]
    gamma_t = -7.0 + 13.0 * t
    gamma_s = -7.0 + 13.0 * (t - 1.0 / TSTEPS)
    alpha = jnp.sqrt(lax.logistic(-gamma_t))
    sigma = jnp.sqrt(lax.logistic(gamma_t))
    snr = 1.0 - jnp.exp(gamma_t - gamma_s)
    sc_ref[...] = jnp.concatenate(
        [counts, alpha, sigma, snr, jnp.zeros((NSEG, 4), jnp.float32)], axis=1)


def _t1a(mask2, eps, t):
    return pl.pallas_call(
        _t1a_body,
        out_shape=[
            jax.ShapeDtypeStruct((NSEG, 16), jnp.float32),
            jax.ShapeDtypeStruct((NSEG, 8), jnp.float32),
        ],
    )(mask2, eps, t)


def _t1b_body(mask_ref, xh_ref, eps_ref, t_ref, cond_ref, wc_ref, wz_ref,
              wt_ref, we1a_ref, we1b_ref, mean_ref, sc_ref,
              ta_ref, tb_ref, hemb_ref, epsc_ref):
    o = _onehot(mask_ref[...], NB)
    sc = sc_ref[...]
    alpha_n = o @ sc[:, 1:2]
    sigma_n = o @ sc[:, 2:3]
    t_n = o @ t_ref[...]
    cmask3 = (lax.broadcasted_iota(jnp.int32, (1, 16), 1) < 3).astype(
        jnp.float32)
    mean_n = (o @ mean_ref[...]) * cmask3
    eps_c = eps_ref[...] - mean_n
    z16 = alpha_n * xh_ref[...] + sigma_n * eps_c
    cond_n = o @ (cond_ref[...] @ wc_ref[...])
    h_emb = _silu(z16 @ wz_ref[...] + t_n @ wt_ref[...] + cond_n)
    pos16 = z16 * cmask3
    pad = jnp.zeros((NB, TW - HID - 16), jnp.float32)
    ta_ref[...] = jnp.concatenate([h_emb @ we1a_ref[...], pos16, pad], axis=1)
    tb_ref[...] = jnp.concatenate([h_emb @ we1b_ref[...], pos16, pad], axis=1)
    hemb_ref[...] = h_emb
    epsc_ref[...] = eps_c


def _t1b(mask2, xh, eps, t, conditions, wc, wz, wt, we1a, we1b, mean, sc):
    nblk = N_NODES // NB
    full = lambda r, c: pl.BlockSpec((r, c), lambda i: (0, 0))
    blk = lambda c: pl.BlockSpec((NB, c), lambda i: (i, 0))
    return pl.pallas_call(
        _t1b_body,
        grid=(nblk,),
        in_specs=[
            blk(1), blk(16), blk(16),
            full(NSEG, 1), full(NSEG, 1), full(1, HID),
            full(16, HID), full(1, HID), full(HID, HID), full(HID, HID),
            full(NSEG, 16), full(NSEG, 8),
        ],
        out_specs=[blk(TW), blk(TW), blk(HID), blk(16)],
        out_shape=[
            jax.ShapeDtypeStruct((N_NODES, TW), jnp.float32),
            jax.ShapeDtypeStruct((N_NODES, TW), jnp.float32),
            jax.ShapeDtypeStruct((N_NODES, HID), jnp.float32),
            jax.ShapeDtypeStruct((N_NODES, 16), jnp.float32),
        ],
    )(mask2, xh, eps, t, conditions, wc, wz, wt, we1a, we1b, mean, sc)


def _s1(ta, tb, src, dst):
    mesh = plsc.VectorSubcoreMesh(core_axis_name="c", subcore_axis_name="s")
    nblk = N_EDGES // GB
    nit = pl.cdiv(nblk, 32)

    @functools.partial(
        pl.kernel, mesh=mesh,
        out_type=[
            jax.ShapeDtypeStruct((N_EDGES, TW), jnp.float32),
            jax.ShapeDtypeStruct((N_EDGES, TW), jnp.float32),
        ],
        scratch_types=[
            pltpu.VMEM((GB,), jnp.int32), pltpu.VMEM((GB,), jnp.int32),
            pltpu.VMEM((GB, TW), jnp.float32), pltpu.VMEM((GB, TW), jnp.float32),
            pltpu.SemaphoreType.DMA, pltpu.SemaphoreType.DMA,
        ])
    def k(ta_hbm, tb_hbm, src_hbm, dst_hbm, asp_hbm, bdp_hbm,
          si, di, ra, rb, sema, semb):
        wid = lax.axis_index("s") * 2 + lax.axis_index("c")

        @pl.loop(0, nit)
        def _(it):
            blk = wid + it * 32

            @pl.when(blk < nblk)
            def _():
                base = blk * GB
                pltpu.sync_copy(src_hbm.at[pl.ds(base, GB)], si)
                pltpu.sync_copy(dst_hbm.at[pl.ds(base, GB)], di)
                ca = pltpu.async_copy(ta_hbm.at[si], ra, sema)
                cb = pltpu.async_copy(tb_hbm.at[di], rb, semb)
                ca.wait()
                cb.wait()
                pltpu.sync_copy(ra, asp_hbm.at[pl.ds(base, GB)])
                pltpu.sync_copy(rb, bdp_hbm.at[pl.ds(base, GB)])

    return k(ta, tb, src, dst)


def _t2_body(a_ref, b_ref, wd_ref, we2_ref, wx_ref, m2a_ref, m2b_ref, rlx_ref):
    a = a_ref[...]
    b = b_ref[...]
    rel = a[:, HID:HID + 16] - b[:, HID:HID + 16]
    d2 = jnp.sum(rel * rel, axis=1, keepdims=True)
    m1 = _silu(a[:, :HID] + b[:, :HID] + d2 @ wd_ref[...])
    m2 = _silu(m1 @ we2_ref[...])
    wx = m2 @ wx_ref[...]
    m2a_ref[...] = m2[:, :128]
    m2b_ref[...] = m2[:, 128:]
    pad128 = (lax.broadcasted_iota(jnp.int32, (16, 128), 0)
              == lax.broadcasted_iota(jnp.int32, (16, 128), 1)).astype(
                  jnp.float32)
    rlx_ref[...] = (rel * wx) @ pad128


def _t2(asp, bdp, wd, we2, wx):
    nblk = N_EDGES // EB
    full = lambda r, c: pl.BlockSpec((r, c), lambda i: (0, 0))
    blk = lambda c: pl.BlockSpec((EB, c), lambda i: (i, 0))
    return pl.pallas_call(
        _t2_body,
        grid=(nblk,),
        in_specs=[blk(TW), blk(TW), full(1, HID), full(HID, HID), full(HID, 1)],
        out_specs=[blk(128), blk(128), blk(128)],
        out_shape=[
            jax.ShapeDtypeStruct((N_EDGES, 128), jnp.float32),
            jax.ShapeDtypeStruct((N_EDGES, 128), jnp.float32),
            jax.ShapeDtypeStruct((N_EDGES, 128), jnp.float32),
        ],
    )(asp, bdp, wd, we2, wx)


def _s2(m2a, m2b, rlx128, dst):
    mesh = plsc.VectorSubcoreMesh(core_axis_name="c", subcore_axis_name="s")
    cp = pltpu.CompilerParams()
    if "needs_layout_passes" in pltpu.CompilerParams.__dataclass_fields__:
        cp = dataclasses.replace(cp, needs_layout_passes=False)
    nblk = N_EDGES // SB
    ROWS = 632
    LAST = N_NODES - 15 * ROWS

    @functools.partial(
        pl.kernel, mesh=mesh, compiler_params=cp,
        out_type=[
            jax.ShapeDtypeStruct((N_NODES, 128), jnp.float32),
            jax.ShapeDtypeStruct((N_NODES, 128), jnp.float32),
            jax.ShapeDtypeStruct((N_NODES, 128), jnp.float32),
            jax.ShapeDtypeStruct((N_NODES, 128), jnp.float32),
        ],
        scratch_types=[
            pltpu.VMEM_SHARED((N_NODES, 128), jnp.float32),
            pltpu.VMEM((8, 128), jnp.float32),
            pltpu.VMEM((SB, 128), jnp.float32),
            pltpu.VMEM((SB,), jnp.int32),
        ])
    def k(m2a_hbm, m2b_hbm, rlx_hbm, dst_hbm, agga_hbm, aggb_hbm, dpa_hbm,
          dpb_hbm, acc, zb, mbuf, idxv):
        c = lax.axis_index("c")
        s = lax.axis_index("s")
        off = s * ROWS
        nz = jnp.where(s == 15, LAST // 8, ROWS // 8)
        zero16 = jnp.zeros((16,), jnp.float32)

        @pl.loop(0, 8)
        def _(i):
            @pl.loop(0, 8)
            def _(j):
                zb[i, pl.ds(j * 16, 16)] = zero16

        def zero_own_rows():
            @pl.loop(0, ROWS // 8)
            def _(i):
                @pl.when(i < nz)
                def _():
                    pltpu.sync_copy(zb, acc.at[pl.ds(off + i * 8, 8)])

        def copy_out(dst_full):
            @pl.when(s < 15)
            def _():
                pltpu.sync_copy(acc.at[pl.ds(off, ROWS)],
                                dst_full.at[pl.ds(off, ROWS)])

            @pl.when(s == 15)
            def _():
                pltpu.sync_copy(acc.at[pl.ds(off, LAST)],
                                dst_full.at[pl.ds(off, LAST)])

        zero_own_rows()
        plsc.subcore_barrier()

        @pl.loop(0, nblk // 16)
        def _(it):
            base = (it * 16 + s) * SB
            pltpu.sync_copy(dst_hbm.at[pl.ds(base, SB)], idxv)

            @pl.when(c == 0)
            def _():
                pltpu.sync_copy(m2a_hbm.at[pl.ds(base, SB)], mbuf)

            @pl.when(c == 1)
            def _():
                pltpu.sync_copy(m2b_hbm.at[pl.ds(base, SB)], mbuf)

            pltpu.sync_copy(mbuf, acc.at[idxv], add=True)

        plsc.subcore_barrier()

        @pl.when(c == 0)
        def _():
            copy_out(agga_hbm)

        @pl.when(c == 1)
        def _():
            copy_out(aggb_hbm)

        zero_own_rows()
        plsc.subcore_barrier()

        @pl.loop(0, 63)
        def _(it):
            myblk = it * 16 + s

            @pl.when(myblk < nblk // 2)
            def _():
                base = (c * (nblk // 2) + myblk) * SB
                pltpu.sync_copy(dst_hbm.at[pl.ds(base, SB)], idxv)
                pltpu.sync_copy(rlx_hbm.at[pl.ds(base, SB)], mbuf)
                pltpu.sync_copy(mbuf, acc.at[idxv], add=True)

        plsc.subcore_barrier()

        @pl.when(c == 0)
        def _():
            copy_out(dpa_hbm)

        @pl.when(c == 1)
        def _():
            copy_out(dpb_hbm)

    return k(m2a, m2b, rlx128, dst)


def _t3a_body(mask_ref, dpa_ref, dpb_ref, sc_ref, out_ref):
    o = _onehot(mask_ref[...], N_NODES)
    dpos = dpa_ref[:, :16] + dpb_ref[:, :16]
    sums = lax.dot_general(o, dpos, (((0,), (0,)), ((), ())))
    out_ref[...] = sums / sc_ref[:, 0:1]


def _t3a(mask2, dpa, dpb, sc):
    return pl.pallas_call(
        _t3a_body,
        out_shape=jax.ShapeDtypeStruct((NSEG, 16), jnp.float32),
    )(mask2, dpa, dpb, sc)


def _t3b_body(mask_ref, hemb_ref, agga_ref, aggb_ref, dpa_ref, dpb_ref,
              epsc_ref, dmean_ref, sc_ref, wht_ref, whb0_ref, whb1_ref,
              wout_ref, out_ref):
    i = pl.program_id(0)
    o = _onehot(mask_ref[...], NB)
    h_new = _silu(hemb_ref[...] @ wht_ref[...] + agga_ref[...] @ whb0_ref[...]
                  + aggb_ref[...] @ whb1_ref[...])
    dpos = dpa_ref[:, :16] + dpb_ref[:, :16]
    net16 = (dpos - o @ dmean_ref[...]) + h_new @ wout_ref[...]
    diff = epsc_ref[...] - net16
    err = jnp.sum(diff * diff, axis=1, keepdims=True)
    part = lax.dot_general(o, err, (((0,), (0,)), ((), ())))

    @pl.when(i == 0)
    def _():
        out_ref[...] = part

    @pl.when(i > 0)
    def _():
        out_ref[...] += part

    @pl.when(i == N_NODES // NB - 1)
    def _():
        out_ref[...] *= sc_ref[:, 3:4]


def _t3b(mask2, hemb, agga, aggb, dpa, dpb, epsc, dmean, sc, wht, whb0, whb1,
         wout_pad):
    nblk = N_NODES // NB
    full = lambda r, c: pl.BlockSpec((r, c), lambda i: (0, 0))
    blk = lambda c: pl.BlockSpec((NB, c), lambda i: (i, 0))
    return pl.pallas_call(
        _t3b_body,
        grid=(nblk,),
        in_specs=[
            blk(1), blk(HID), blk(128), blk(128), blk(128), blk(128), blk(16),
            full(NSEG, 16), full(NSEG, 8),
            full(HID, HID), full(128, HID), full(128, HID), full(HID, 16),
        ],
        out_specs=pl.BlockSpec((NSEG, 1), lambda i: (0, 0)),
        out_shape=jax.ShapeDtypeStruct((NSEG, 1), jnp.float32),
    )(mask2, hemb, agga, aggb, dpa, dpb, epsc, dmean, sc, wht, whb0, whb1,
      wout_pad)


def kernel(pos, h, eps, t, conditions, W_in, Wc, We1, We2, Wx, Wh, Wout,
           combined_mask, edge_index):
    f32 = jnp.float32
    mask2 = combined_mask.reshape(N_NODES, 1).astype(jnp.int32)
    src = edge_index[0].astype(jnp.int32)
    dst = edge_index[1].astype(jnp.int32)
    xh = jnp.concatenate([pos, h], axis=1)
    wz = jnp.concatenate([jnp.zeros((3, HID), f32), W_in[:13]], axis=0)
    wt = W_in[13:14]
    we1a = We1[:HID]
    we1b = We1[HID:2 * HID]
    wd = We1[2 * HID:2 * HID + 1]
    wht = Wh[:HID]
    whb0 = Wh[HID:HID + 128]
    whb1 = Wh[HID + 128:]
    wout_pad = jnp.concatenate([jnp.zeros((HID, 3), f32), Wout], axis=1)

    mean, sc = _t1a(mask2, eps, t)
    ta, tb, hemb, epsc = _t1b(mask2, xh, eps, t, conditions, Wc, wz, wt,
                              we1a, we1b, mean, sc)
    asp, bdp = _s1(ta, tb, src, dst)
    m2a, m2b, rlx = _t2(asp, bdp, wd, We2, Wx)
    agga, aggb, dpa, dpb = _s2(m2a, m2b, rlx, dst)
    dmean = _t3a(mask2, dpa, dpb, sc)
    err = _t3b(mask2, hemb, agga, aggb, dpa, dpb, epsc, dmean, sc, wht, whb0,
               whb1, wout_pad)
    return err.reshape(NSEG)


# speedup vs baseline: 3.1040x; 1.0792x over previous
import dataclasses
import functools

import jax
import jax.numpy as jnp
from jax import lax
from jax.experimental import pallas as pl
from jax.experimental.pallas import tpu as pltpu
from jax.experimental.pallas import tpu_sc as plsc

N_NODES = 10000
N_EDGES = 160000
NSEG = 256
HID = 256
TSTEPS = 1000.0

NB = 2000
EB = 1280
GB = 128
SB = 80
TW = 256


def _silu(x):
    return x * lax.logistic(x)


def _onehot(m, rows):
    return (m == lax.broadcasted_iota(jnp.int32, (rows, NSEG), 1)).astype(
        jnp.float32)


def _t1a_body(mask_ref, eps_ref, t_ref, mean_ref, sc_ref):
    o = _onehot(mask_ref[...], N_NODES)
    ones = jnp.ones((N_NODES, 1), jnp.float32)
    cdims = (((0,), (0,)), ((), ()))
    counts = jnp.maximum(lax.dot_general(o, ones, cdims), 1.0)
    sums = lax.dot_general(o, eps_ref[...], cdims)
    mean_ref[...] = sums / counts
    t = t_ref[...]
    gamma_t = -7.0 + 13.0 * t
    gamma_s = -7.0 + 13.0 * (t - 1.0 / TSTEPS)
    alpha = jnp.sqrt(lax.logistic(-gamma_t))
    sigma = jnp.sqrt(lax.logistic(gamma_t))
    snr = 1.0 - jnp.exp(gamma_t - gamma_s)
    sc_ref[...] = jnp.concatenate(
        [counts, alpha, sigma, snr, jnp.zeros((NSEG, 4), jnp.float32)], axis=1)


def _t1a(mask2, eps, t):
    return pl.pallas_call(
        _t1a_body,
        out_shape=[
            jax.ShapeDtypeStruct((NSEG, 16), jnp.float32),
            jax.ShapeDtypeStruct((NSEG, 8), jnp.float32),
        ],
    )(mask2, eps, t)


def _t1b_body(mask_ref, xh_ref, eps_ref, t_ref, cond_ref, wc_ref, wz_ref,
              wt_ref, we1a_ref, we1b_ref, mean_ref, sc_ref,
              ta_ref, tb_ref, pos_ref, hemb_ref, epsc_ref):
    o = _onehot(mask_ref[...], NB)
    sc = sc_ref[...]
    alpha_n = o @ sc[:, 1:2]
    sigma_n = o @ sc[:, 2:3]
    t_n = o @ t_ref[...]
    cmask3 = (lax.broadcasted_iota(jnp.int32, (1, 16), 1) < 3).astype(
        jnp.float32)
    mean_n = (o @ mean_ref[...]) * cmask3
    eps_c = eps_ref[...] - mean_n
    z16 = alpha_n * xh_ref[...] + sigma_n * eps_c
    cond_n = o @ (cond_ref[...] @ wc_ref[...])
    h_emb = _silu(z16 @ wz_ref[...] + t_n @ wt_ref[...] + cond_n)
    cmask4 = (lax.broadcasted_iota(jnp.int32, (1, 4), 1) < 3).astype(
        jnp.float32)
    ta_ref[...] = h_emb @ we1a_ref[...]
    tb_ref[...] = h_emb @ we1b_ref[...]
    pos_ref[...] = z16[:, 0:4] * cmask4
    hemb_ref[...] = h_emb
    epsc_ref[...] = eps_c


def _t1b(mask2, xh, eps, t, conditions, wc, wz, wt, we1a, we1b, mean, sc):
    nblk = N_NODES // NB
    full = lambda r, c: pl.BlockSpec((r, c), lambda i: (0, 0))
    blk = lambda c: pl.BlockSpec((NB, c), lambda i: (i, 0))
    return pl.pallas_call(
        _t1b_body,
        grid=(nblk,),
        in_specs=[
            blk(1), blk(16), blk(16),
            full(NSEG, 1), full(NSEG, 1), full(1, HID),
            full(16, HID), full(1, HID), full(HID, HID), full(HID, HID),
            full(NSEG, 16), full(NSEG, 8),
        ],
        out_specs=[blk(TW), blk(TW), blk(4), blk(HID), blk(16)],
        out_shape=[
            jax.ShapeDtypeStruct((N_NODES, TW), jnp.float32),
            jax.ShapeDtypeStruct((N_NODES, TW), jnp.float32),
            jax.ShapeDtypeStruct((N_NODES, 4), jnp.float32),
            jax.ShapeDtypeStruct((N_NODES, HID), jnp.float32),
            jax.ShapeDtypeStruct((N_NODES, 16), jnp.float32),
        ],
    )(mask2, xh, eps, t, conditions, wc, wz, wt, we1a, we1b, mean, sc)


def _s1(ta, tb, pos4, src, dst):
    mesh = plsc.VectorSubcoreMesh(core_axis_name="c", subcore_axis_name="s")
    cp = pltpu.CompilerParams()
    if "needs_layout_passes" in pltpu.CompilerParams.__dataclass_fields__:
        cp = dataclasses.replace(cp, needs_layout_passes=False)
    nblk = N_EDGES // GB
    nit = pl.cdiv(nblk, 32)

    @functools.partial(
        pl.kernel, mesh=mesh, compiler_params=cp,
        out_type=[
            jax.ShapeDtypeStruct((N_EDGES, TW), jnp.float32),
            jax.ShapeDtypeStruct((N_EDGES, TW), jnp.float32),
            jax.ShapeDtypeStruct((8, N_EDGES), jnp.float32),
        ],
        scratch_types=[
            pltpu.VMEM((GB,), jnp.int32), pltpu.VMEM((GB,), jnp.int32),
            pltpu.VMEM((GB, TW), jnp.float32), pltpu.VMEM((GB, TW), jnp.float32),
            pltpu.VMEM((4 * N_NODES,), jnp.float32),
            pltpu.VMEM((8, GB), jnp.float32),
            pltpu.SemaphoreType.DMA, pltpu.SemaphoreType.DMA,
        ])
    def k(ta_hbm, tb_hbm, pos_hbm, src_hbm, dst_hbm, asp_hbm, bdp_hbm,
          geo_hbm, si, di, ra, rb, posv, gbuf, sema, semb):
        wid = lax.axis_index("s") * 2 + lax.axis_index("c")
        pltpu.sync_copy(pos_hbm, posv)
        zero16 = jnp.zeros((16,), jnp.float32)

        @pl.loop(0, 8)
        def _(g):
            gbuf[3, pl.ds(g * 16, 16)] = zero16
            gbuf[4, pl.ds(g * 16, 16)] = zero16
            gbuf[5, pl.ds(g * 16, 16)] = zero16
            gbuf[6, pl.ds(g * 16, 16)] = zero16
            gbuf[7, pl.ds(g * 16, 16)] = zero16

        @pl.loop(0, nit)
        def _(it):
            blk = wid + it * 32

            @pl.when(blk < nblk)
            def _():
                base = blk * GB
                pltpu.sync_copy(src_hbm.at[pl.ds(base, GB)], si)
                pltpu.sync_copy(dst_hbm.at[pl.ds(base, GB)], di)
                ca = pltpu.async_copy(ta_hbm.at[si], ra, sema)
                cb = pltpu.async_copy(tb_hbm.at[di], rb, semb)

                @pl.loop(0, GB // 16)
                def _(g):
                    sidx = si[pl.ds(g * 16, 16)] * 4
                    didx = di[pl.ds(g * 16, 16)] * 4
                    for kc in range(3):
                        ps = plsc.load_gather(posv, [sidx + kc])
                        pd = plsc.load_gather(posv, [didx + kc])
                        gbuf[kc, pl.ds(g * 16, 16)] = ps - pd

                ca.wait()
                cb.wait()
                pltpu.sync_copy(ra, asp_hbm.at[pl.ds(base, GB)])
                pltpu.sync_copy(rb, bdp_hbm.at[pl.ds(base, GB)])
                pltpu.sync_copy(gbuf, geo_hbm.at[:, pl.ds(base, GB)])

    return k(ta, tb, pos4, src, dst)


def _t2_body(a_ref, b_ref, g_ref, wd_ref, we2_ref, wx_ref, m2a_ref, m2b_ref,
             rlx_ref):
    a = a_ref[...]
    b = b_ref[...]
    eye8 = (lax.broadcasted_iota(jnp.int32, (8, 8), 0)
            == lax.broadcasted_iota(jnp.int32, (8, 8), 1)).astype(jnp.float32)
    rel = lax.dot_general(g_ref[...], eye8, (((0,), (0,)), ((), ())))
    d2 = jnp.sum(rel * rel, axis=1, keepdims=True)
    m1 = _silu(a + b + d2 @ wd_ref[...])
    m2 = _silu(m1 @ we2_ref[...])
    wx = m2 @ wx_ref[...]
    m2a_ref[...] = m2[:, :128]
    m2b_ref[...] = m2[:, 128:]
    pad128 = (lax.broadcasted_iota(jnp.int32, (8, 128), 0)
              == lax.broadcasted_iota(jnp.int32, (8, 128), 1)).astype(
                  jnp.float32)
    rlx_ref[...] = (rel * wx) @ pad128


def _t2(asp, bdp, geo, wd, we2, wx):
    nblk = N_EDGES // EB
    full = lambda r, c: pl.BlockSpec((r, c), lambda i: (0, 0))
    blk = lambda c: pl.BlockSpec((EB, c), lambda i: (i, 0))
    return pl.pallas_call(
        _t2_body,
        grid=(nblk,),
        in_specs=[blk(TW), blk(TW), pl.BlockSpec((8, EB), lambda i: (0, i)),
                  full(1, HID), full(HID, HID), full(HID, 1)],
        out_specs=[blk(128), blk(128), blk(128)],
        out_shape=[
            jax.ShapeDtypeStruct((N_EDGES, 128), jnp.float32),
            jax.ShapeDtypeStruct((N_EDGES, 128), jnp.float32),
            jax.ShapeDtypeStruct((N_EDGES, 128), jnp.float32),
        ],
    )(asp, bdp, geo, wd, we2, wx)


def _s2(m2a, m2b, rlx128, dst):
    mesh = plsc.VectorSubcoreMesh(core_axis_name="c", subcore_axis_name="s")
    cp = pltpu.CompilerParams()
    if "needs_layout_passes" in pltpu.CompilerParams.__dataclass_fields__:
        cp = dataclasses.replace(cp, needs_layout_passes=False)
    nblk = N_EDGES // SB
    ROWS = 632
    LAST = N_NODES - 15 * ROWS

    @functools.partial(
        pl.kernel, mesh=mesh, compiler_params=cp,
        out_type=[
            jax.ShapeDtypeStruct((N_NODES, 128), jnp.float32),
            jax.ShapeDtypeStruct((N_NODES, 128), jnp.float32),
            jax.ShapeDtypeStruct((N_NODES, 128), jnp.float32),
            jax.ShapeDtypeStruct((N_NODES, 128), jnp.float32),
        ],
        scratch_types=[
            pltpu.VMEM_SHARED((N_NODES, 128), jnp.float32),
            pltpu.VMEM((8, 128), jnp.float32),
            pltpu.VMEM((SB, 128), jnp.float32),
            pltpu.VMEM((SB,), jnp.int32),
        ])
    def k(m2a_hbm, m2b_hbm, rlx_hbm, dst_hbm, agga_hbm, aggb_hbm, dpa_hbm,
          dpb_hbm, acc, zb, mbuf, idxv):
        c = lax.axis_index("c")
        s = lax.axis_index("s")
        off = s * ROWS
        nz = jnp.where(s == 15, LAST // 8, ROWS // 8)
        zero16 = jnp.zeros((16,), jnp.float32)

        @pl.loop(0, 8)
        def _(i):
            @pl.loop(0, 8)
            def _(j):
                zb[i, pl.ds(j * 16, 16)] = zero16

        def zero_own_rows():
            @pl.loop(0, ROWS // 8)
            def _(i):
                @pl.when(i < nz)
                def _():
                    pltpu.sync_copy(zb, acc.at[pl.ds(off + i * 8, 8)])

        def copy_out(dst_full):
            @pl.when(s < 15)
            def _():
                pltpu.sync_copy(acc.at[pl.ds(off, ROWS)],
                                dst_full.at[pl.ds(off, ROWS)])

            @pl.when(s == 15)
            def _():
                pltpu.sync_copy(acc.at[pl.ds(off, LAST)],
                                dst_full.at[pl.ds(off, LAST)])

        zero_own_rows()
        plsc.subcore_barrier()

        @pl.loop(0, nblk // 16)
        def _(it):
            base = (it * 16 + s) * SB
            pltpu.sync_copy(dst_hbm.at[pl.ds(base, SB)], idxv)

            @pl.when(c == 0)
            def _():
                pltpu.sync_copy(m2a_hbm.at[pl.ds(base, SB)], mbuf)

            @pl.when(c == 1)
            def _():
                pltpu.sync_copy(m2b_hbm.at[pl.ds(base, SB)], mbuf)

            pltpu.sync_copy(mbuf, acc.at[idxv], add=True)

        plsc.subcore_barrier()

        @pl.when(c == 0)
        def _():
            copy_out(agga_hbm)

        @pl.when(c == 1)
        def _():
            copy_out(aggb_hbm)

        zero_own_rows()
        plsc.subcore_barrier()

        @pl.loop(0, 63)
        def _(it):
            myblk = it * 16 + s

            @pl.when(myblk < nblk // 2)
            def _():
                base = (c * (nblk // 2) + myblk) * SB
                pltpu.sync_copy(dst_hbm.at[pl.ds(base, SB)], idxv)
                pltpu.sync_copy(rlx_hbm.at[pl.ds(base, SB)], mbuf)
                pltpu.sync_copy(mbuf, acc.at[idxv], add=True)

        plsc.subcore_barrier()

        @pl.when(c == 0)
        def _():
            copy_out(dpa_hbm)

        @pl.when(c == 1)
        def _():
            copy_out(dpb_hbm)

    return k(m2a, m2b, rlx128, dst)


def _t3a_body(mask_ref, dpa_ref, dpb_ref, sc_ref, out_ref):
    o = _onehot(mask_ref[...], N_NODES)
    dpos = dpa_ref[:, :16] + dpb_ref[:, :16]
    sums = lax.dot_general(o, dpos, (((0,), (0,)), ((), ())))
    out_ref[...] = sums / sc_ref[:, 0:1]


def _t3a(mask2, dpa, dpb, sc):
    return pl.pallas_call(
        _t3a_body,
        out_shape=jax.ShapeDtypeStruct((NSEG, 16), jnp.float32),
    )(mask2, dpa, dpb, sc)


def _t3b_body(mask_ref, hemb_ref, agga_ref, aggb_ref, dpa_ref, dpb_ref,
              epsc_ref, dmean_ref, sc_ref, wht_ref, whb0_ref, whb1_ref,
              wout_ref, out_ref):
    i = pl.program_id(0)
    o = _onehot(mask_ref[...], NB)
    h_new = _silu(hemb_ref[...] @ wht_ref[...] + agga_ref[...] @ whb0_ref[...]
                  + aggb_ref[...] @ whb1_ref[...])
    dpos = dpa_ref[:, :16] + dpb_ref[:, :16]
    net16 = (dpos - o @ dmean_ref[...]) + h_new @ wout_ref[...]
    diff = epsc_ref[...] - net16
    err = jnp.sum(diff * diff, axis=1, keepdims=True)
    part = lax.dot_general(o, err, (((0,), (0,)), ((), ())))

    @pl.when(i == 0)
    def _():
        out_ref[...] = part

    @pl.when(i > 0)
    def _():
        out_ref[...] += part

    @pl.when(i == N_NODES // NB - 1)
    def _():
        out_ref[...] *= sc_ref[:, 3:4]


def _t3b(mask2, hemb, agga, aggb, dpa, dpb, epsc, dmean, sc, wht, whb0, whb1,
         wout_pad):
    nblk = N_NODES // NB
    full = lambda r, c: pl.BlockSpec((r, c), lambda i: (0, 0))
    blk = lambda c: pl.BlockSpec((NB, c), lambda i: (i, 0))
    return pl.pallas_call(
        _t3b_body,
        grid=(nblk,),
        in_specs=[
            blk(1), blk(HID), blk(128), blk(128), blk(128), blk(128), blk(16),
            full(NSEG, 16), full(NSEG, 8),
            full(HID, HID), full(128, HID), full(128, HID), full(HID, 16),
        ],
        out_specs=pl.BlockSpec((NSEG, 1), lambda i: (0, 0)),
        out_shape=jax.ShapeDtypeStruct((NSEG, 1), jnp.float32),
    )(mask2, hemb, agga, aggb, dpa, dpb, epsc, dmean, sc, wht, whb0, whb1,
      wout_pad)


def kernel(pos, h, eps, t, conditions, W_in, Wc, We1, We2, Wx, Wh, Wout,
           combined_mask, edge_index):
    f32 = jnp.float32
    mask2 = combined_mask.reshape(N_NODES, 1).astype(jnp.int32)
    src = edge_index[0].astype(jnp.int32)
    dst = edge_index[1].astype(jnp.int32)
    xh = jnp.concatenate([pos, h], axis=1)
    wz = jnp.concatenate([jnp.zeros((3, HID), f32), W_in[:13]], axis=0)
    wt = W_in[13:14]
    we1a = We1[:HID]
    we1b = We1[HID:2 * HID]
    wd = We1[2 * HID:2 * HID + 1]
    wht = Wh[:HID]
    whb0 = Wh[HID:HID + 128]
    whb1 = Wh[HID + 128:]
    wout_pad = jnp.concatenate([jnp.zeros((HID, 3), f32), Wout], axis=1)

    mean, sc = _t1a(mask2, eps, t)
    ta, tb, pos4, hemb, epsc = _t1b(mask2, xh, eps, t, conditions, Wc, wz, wt,
                                    we1a, we1b, mean, sc)
    asp, bdp, geo = _s1(ta, tb, pos4.reshape(4 * N_NODES), src, dst)
    m2a, m2b, rlx = _t2(asp, bdp, geo, wd, We2, Wx)
    agga, aggb, dpa, dpb = _s2(m2a, m2b, rlx, dst)
    dmean = _t3a(mask2, dpa, dpb, sc)
    err = _t3b(mask2, hemb, agga, aggb, dpa, dpb, epsc, dmean, sc, wht, whb0,
               whb1, wout_pad)
    return err.reshape(NSEG)


# speedup vs baseline: 3.3356x; 1.0746x over previous
import dataclasses
import functools

import jax
import jax.numpy as jnp
from jax import lax
from jax.experimental import pallas as pl
from jax.experimental.pallas import tpu as pltpu
from jax.experimental.pallas import tpu_sc as plsc

N_NODES = 10000
N_EDGES = 160000
NSEG = 256
HID = 256
TSTEPS = 1000.0

NB = 2000
EB = 1280
GB = 128
SB = 128
TW = 256


def _silu(x):
    return x * lax.logistic(x)


def _onehot(m, rows):
    return (m == lax.broadcasted_iota(jnp.int32, (rows, NSEG), 1)).astype(
        jnp.float32)


def _t1a_body(mask_ref, eps_ref, t_ref, mean_ref, sc_ref):
    o = _onehot(mask_ref[...], N_NODES)
    ones = jnp.ones((N_NODES, 1), jnp.float32)
    cdims = (((0,), (0,)), ((), ()))
    counts = jnp.maximum(lax.dot_general(o, ones, cdims), 1.0)
    sums = lax.dot_general(o, eps_ref[...], cdims)
    mean_ref[...] = sums / counts
    t = t_ref[...]
    gamma_t = -7.0 + 13.0 * t
    gamma_s = -7.0 + 13.0 * (t - 1.0 / TSTEPS)
    alpha = jnp.sqrt(lax.logistic(-gamma_t))
    sigma = jnp.sqrt(lax.logistic(gamma_t))
    snr = 1.0 - jnp.exp(gamma_t - gamma_s)
    sc_ref[...] = jnp.concatenate(
        [counts, alpha, sigma, snr, jnp.zeros((NSEG, 4), jnp.float32)], axis=1)


def _t1a(mask2, eps, t):
    return pl.pallas_call(
        _t1a_body,
        out_shape=[
            jax.ShapeDtypeStruct((NSEG, 16), jnp.float32),
            jax.ShapeDtypeStruct((NSEG, 8), jnp.float32),
        ],
    )(mask2, eps, t)


def _t1b_body(mask_ref, xh_ref, eps_ref, t_ref, cond_ref, wc_ref, wz_ref,
              wt_ref, we1a_ref, we1b_ref, mean_ref, sc_ref,
              ta_ref, tb_ref, pos_ref, hemb_ref, epsc_ref):
    o = _onehot(mask_ref[...], NB)
    sc = sc_ref[...]
    alpha_n = o @ sc[:, 1:2]
    sigma_n = o @ sc[:, 2:3]
    t_n = o @ t_ref[...]
    cmask3 = (lax.broadcasted_iota(jnp.int32, (1, 16), 1) < 3).astype(
        jnp.float32)
    mean_n = (o @ mean_ref[...]) * cmask3
    eps_c = eps_ref[...] - mean_n
    z16 = alpha_n * xh_ref[...] + sigma_n * eps_c
    cond_n = o @ (cond_ref[...] @ wc_ref[...])
    h_emb = _silu(z16 @ wz_ref[...] + t_n @ wt_ref[...] + cond_n)
    cmask4 = (lax.broadcasted_iota(jnp.int32, (1, 4), 1) < 3).astype(
        jnp.float32)
    ta_ref[...] = h_emb @ we1a_ref[...]
    tb_ref[...] = h_emb @ we1b_ref[...]
    pos_ref[...] = z16[:, 0:4] * cmask4
    hemb_ref[...] = h_emb
    epsc_ref[...] = eps_c


def _t1b(mask2, xh, eps, t, conditions, wc, wz, wt, we1a, we1b, mean, sc):
    nblk = N_NODES // NB
    full = lambda r, c: pl.BlockSpec((r, c), lambda i: (0, 0))
    blk = lambda c: pl.BlockSpec((NB, c), lambda i: (i, 0))
    return pl.pallas_call(
        _t1b_body,
        grid=(nblk,),
        in_specs=[
            blk(1), blk(16), blk(16),
            full(NSEG, 1), full(NSEG, 1), full(1, HID),
            full(16, HID), full(1, HID), full(HID, HID), full(HID, HID),
            full(NSEG, 16), full(NSEG, 8),
        ],
        out_specs=[blk(TW), blk(TW), blk(4), blk(HID), blk(16)],
        out_shape=[
            jax.ShapeDtypeStruct((N_NODES, TW), jnp.float32),
            jax.ShapeDtypeStruct((N_NODES, TW), jnp.float32),
            jax.ShapeDtypeStruct((N_NODES, 4), jnp.float32),
            jax.ShapeDtypeStruct((N_NODES, HID), jnp.float32),
            jax.ShapeDtypeStruct((N_NODES, 16), jnp.float32),
        ],
    )(mask2, xh, eps, t, conditions, wc, wz, wt, we1a, we1b, mean, sc)


def _s1(ta, tb, pos4, src, dst):
    mesh = plsc.VectorSubcoreMesh(core_axis_name="c", subcore_axis_name="s")
    cp = pltpu.CompilerParams()
    if "needs_layout_passes" in pltpu.CompilerParams.__dataclass_fields__:
        cp = dataclasses.replace(cp, needs_layout_passes=False)
    nblk = N_EDGES // GB
    nit = pl.cdiv(nblk, 32)

    @functools.partial(
        pl.kernel, mesh=mesh, compiler_params=cp,
        out_type=[
            jax.ShapeDtypeStruct((N_EDGES, TW), jnp.float32),
            jax.ShapeDtypeStruct((N_EDGES, TW), jnp.float32),
            jax.ShapeDtypeStruct((8, N_EDGES), jnp.float32),
        ],
        scratch_types=[
            pltpu.VMEM((GB,), jnp.int32), pltpu.VMEM((GB,), jnp.int32),
            pltpu.VMEM((GB, TW), jnp.float32), pltpu.VMEM((GB, TW), jnp.float32),
            pltpu.VMEM((4 * N_NODES,), jnp.float32),
            pltpu.VMEM((8, GB), jnp.float32),
            pltpu.SemaphoreType.DMA, pltpu.SemaphoreType.DMA,
            pltpu.SemaphoreType.DMA, pltpu.SemaphoreType.DMA,
            pltpu.SemaphoreType.DMA, pltpu.SemaphoreType.DMA,
        ])
    def k(ta_hbm, tb_hbm, pos_hbm, src_hbm, dst_hbm, asp_hbm, bdp_hbm,
          geo_hbm, si, di, ra, rb, posv, gbuf, sema, semb, sia, sib, swa,
          swb):
        wid = lax.axis_index("s") * 2 + lax.axis_index("c")
        pltpu.sync_copy(pos_hbm, posv)
        zero16 = jnp.zeros((16,), jnp.float32)

        @pl.loop(0, 8)
        def _(g):
            gbuf[3, pl.ds(g * 16, 16)] = zero16
            gbuf[4, pl.ds(g * 16, 16)] = zero16
            gbuf[5, pl.ds(g * 16, 16)] = zero16
            gbuf[6, pl.ds(g * 16, 16)] = zero16
            gbuf[7, pl.ds(g * 16, 16)] = zero16

        @pl.loop(0, nit)
        def _(it):
            blk = wid + it * 32

            @pl.when(blk < nblk)
            def _():
                base = blk * GB
                pltpu.sync_copy(src_hbm.at[pl.ds(base, GB)], si)
                pltpu.sync_copy(dst_hbm.at[pl.ds(base, GB)], di)
                ca = pltpu.async_copy(ta_hbm.at[si], ra, sema)
                cb = pltpu.async_copy(tb_hbm.at[di], rb, semb)

                @pl.loop(0, GB // 16)
                def _(g):
                    sidx = si[pl.ds(g * 16, 16)] * 4
                    didx = di[pl.ds(g * 16, 16)] * 4
                    for kc in range(3):
                        ps = plsc.load_gather(posv, [sidx + kc])
                        pd = plsc.load_gather(posv, [didx + kc])
                        gbuf[kc, pl.ds(g * 16, 16)] = ps - pd

                ca.wait()
                cb.wait()
                pltpu.sync_copy(ra, asp_hbm.at[pl.ds(base, GB)])
                pltpu.sync_copy(rb, bdp_hbm.at[pl.ds(base, GB)])
                pltpu.sync_copy(gbuf, geo_hbm.at[:, pl.ds(base, GB)])

    return k(ta, tb, pos4, src, dst)


def _t2_body(a_ref, b_ref, g_ref, wd_ref, we2_ref, wx_ref, m2a_ref, m2b_ref,
             rlx_ref):
    a = a_ref[...]
    b = b_ref[...]
    eye8 = (lax.broadcasted_iota(jnp.int32, (8, 8), 0)
            == lax.broadcasted_iota(jnp.int32, (8, 8), 1)).astype(jnp.float32)
    rel = lax.dot_general(g_ref[...], eye8, (((0,), (0,)), ((), ())))
    d2 = jnp.sum(rel * rel, axis=1, keepdims=True)
    m1 = _silu(a + b + d2 @ wd_ref[...])
    m2 = _silu(m1 @ we2_ref[...])
    wx = m2 @ wx_ref[...]
    m2a_ref[...] = m2[:, :128]
    m2b_ref[...] = m2[:, 128:]
    pad128 = (lax.broadcasted_iota(jnp.int32, (8, 128), 0)
              == lax.broadcasted_iota(jnp.int32, (8, 128), 1)).astype(
                  jnp.float32)
    rlx_ref[...] = (rel * wx) @ pad128


def _t2(asp, bdp, geo, wd, we2, wx):
    nblk = N_EDGES // EB
    full = lambda r, c: pl.BlockSpec((r, c), lambda i: (0, 0))
    blk = lambda c: pl.BlockSpec((EB, c), lambda i: (i, 0))
    return pl.pallas_call(
        _t2_body,
        grid=(nblk,),
        in_specs=[blk(TW), blk(TW), pl.BlockSpec((8, EB), lambda i: (0, i)),
                  full(1, HID), full(HID, HID), full(HID, 1)],
        out_specs=[blk(128), blk(128), blk(128)],
        out_shape=[
            jax.ShapeDtypeStruct((N_EDGES, 128), jnp.float32),
            jax.ShapeDtypeStruct((N_EDGES, 128), jnp.float32),
            jax.ShapeDtypeStruct((N_EDGES, 128), jnp.float32),
        ],
    )(asp, bdp, geo, wd, we2, wx)


def _s2(m2a, m2b, rlx128, dst):
    mesh = plsc.VectorSubcoreMesh(core_axis_name="c", subcore_axis_name="s")
    cp = pltpu.CompilerParams()
    if "needs_layout_passes" in pltpu.CompilerParams.__dataclass_fields__:
        cp = dataclasses.replace(cp, needs_layout_passes=False)
    nblk = N_EDGES // SB
    ROWS = 632
    LAST = N_NODES - 15 * ROWS

    @functools.partial(
        pl.kernel, mesh=mesh, compiler_params=cp,
        out_type=[
            jax.ShapeDtypeStruct((N_NODES, 128), jnp.float32),
            jax.ShapeDtypeStruct((N_NODES, 128), jnp.float32),
            jax.ShapeDtypeStruct((N_NODES, 128), jnp.float32),
            jax.ShapeDtypeStruct((N_NODES, 128), jnp.float32),
        ],
        scratch_types=[
            pltpu.VMEM_SHARED((N_NODES, 128), jnp.float32),
            pltpu.VMEM((8, 128), jnp.float32),
            pltpu.VMEM((SB, 128), jnp.float32),
            pltpu.VMEM((SB,), jnp.int32),
        ])
    def k(m2a_hbm, m2b_hbm, rlx_hbm, dst_hbm, agga_hbm, aggb_hbm, dpa_hbm,
          dpb_hbm, acc, zb, mbuf, idxv):
        c = lax.axis_index("c")
        s = lax.axis_index("s")
        off = s * ROWS
        nz = jnp.where(s == 15, LAST // 8, ROWS // 8)
        zero16 = jnp.zeros((16,), jnp.float32)

        @pl.loop(0, 8)
        def _(i):
            @pl.loop(0, 8)
            def _(j):
                zb[i, pl.ds(j * 16, 16)] = zero16

        def zero_own_rows():
            @pl.loop(0, ROWS // 8)
            def _(i):
                @pl.when(i < nz)
                def _():
                    pltpu.sync_copy(zb, acc.at[pl.ds(off + i * 8, 8)])

        def copy_out(dst_full):
            @pl.when(s < 15)
            def _():
                pltpu.sync_copy(acc.at[pl.ds(off, ROWS)],
                                dst_full.at[pl.ds(off, ROWS)])

            @pl.when(s == 15)
            def _():
                pltpu.sync_copy(acc.at[pl.ds(off, LAST)],
                                dst_full.at[pl.ds(off, LAST)])

        zero_own_rows()
        plsc.subcore_barrier()

        @pl.loop(0, pl.cdiv(nblk, 16))
        def _(it):
            blk1 = it * 16 + s

            @pl.when(blk1 < nblk)
            def _():
                base = blk1 * SB
                pltpu.sync_copy(dst_hbm.at[pl.ds(base, SB)], idxv)

                @pl.when(c == 0)
                def _():
                    pltpu.sync_copy(m2a_hbm.at[pl.ds(base, SB)], mbuf)

                @pl.when(c == 1)
                def _():
                    pltpu.sync_copy(m2b_hbm.at[pl.ds(base, SB)], mbuf)

                pltpu.sync_copy(mbuf, acc.at[idxv], add=True)

        plsc.subcore_barrier()

        @pl.when(c == 0)
        def _():
            copy_out(agga_hbm)

        @pl.when(c == 1)
        def _():
            copy_out(aggb_hbm)

        zero_own_rows()
        plsc.subcore_barrier()

        @pl.loop(0, pl.cdiv(nblk // 2, 16))
        def _(it):
            myblk = it * 16 + s

            @pl.when(myblk < nblk // 2)
            def _():
                base = (c * (nblk // 2) + myblk) * SB
                pltpu.sync_copy(dst_hbm.at[pl.ds(base, SB)], idxv)
                pltpu.sync_copy(rlx_hbm.at[pl.ds(base, SB)], mbuf)
                pltpu.sync_copy(mbuf, acc.at[idxv], add=True)

        plsc.subcore_barrier()

        @pl.when(c == 0)
        def _():
            copy_out(dpa_hbm)

        @pl.when(c == 1)
        def _():
            copy_out(dpb_hbm)

    return k(m2a, m2b, rlx128, dst)


def _t3a_body(mask_ref, dpa_ref, dpb_ref, sc_ref, out_ref):
    o = _onehot(mask_ref[...], N_NODES)
    dpos = dpa_ref[:, :16] + dpb_ref[:, :16]
    sums = lax.dot_general(o, dpos, (((0,), (0,)), ((), ())))
    out_ref[...] = sums / sc_ref[:, 0:1]


def _t3a(mask2, dpa, dpb, sc):
    return pl.pallas_call(
        _t3a_body,
        out_shape=jax.ShapeDtypeStruct((NSEG, 16), jnp.float32),
    )(mask2, dpa, dpb, sc)


def _t3b_body(mask_ref, hemb_ref, agga_ref, aggb_ref, dpa_ref, dpb_ref,
              epsc_ref, dmean_ref, sc_ref, wht_ref, whb0_ref, whb1_ref,
              wout_ref, out_ref):
    i = pl.program_id(0)
    o = _onehot(mask_ref[...], NB)
    h_new = _silu(hemb_ref[...] @ wht_ref[...] + agga_ref[...] @ whb0_ref[...]
                  + aggb_ref[...] @ whb1_ref[...])
    dpos = dpa_ref[:, :16] + dpb_ref[:, :16]
    net16 = (dpos - o @ dmean_ref[...]) + h_new @ wout_ref[...]
    diff = epsc_ref[...] - net16
    err = jnp.sum(diff * diff, axis=1, keepdims=True)
    part = lax.dot_general(o, err, (((0,), (0,)), ((), ())))

    @pl.when(i == 0)
    def _():
        out_ref[...] = part

    @pl.when(i > 0)
    def _():
        out_ref[...] += part

    @pl.when(i == N_NODES // NB - 1)
    def _():
        out_ref[...] *= sc_ref[:, 3:4]


def _t3b(mask2, hemb, agga, aggb, dpa, dpb, epsc, dmean, sc, wht, whb0, whb1,
         wout_pad):
    nblk = N_NODES // NB
    full = lambda r, c: pl.BlockSpec((r, c), lambda i: (0, 0))
    blk = lambda c: pl.BlockSpec((NB, c), lambda i: (i, 0))
    return pl.pallas_call(
        _t3b_body,
        grid=(nblk,),
        in_specs=[
            blk(1), blk(HID), blk(128), blk(128), blk(128), blk(128), blk(16),
            full(NSEG, 16), full(NSEG, 8),
            full(HID, HID), full(128, HID), full(128, HID), full(HID, 16),
        ],
        out_specs=pl.BlockSpec((NSEG, 1), lambda i: (0, 0)),
        out_shape=jax.ShapeDtypeStruct((NSEG, 1), jnp.float32),
    )(mask2, hemb, agga, aggb, dpa, dpb, epsc, dmean, sc, wht, whb0, whb1,
      wout_pad)


def kernel(pos, h, eps, t, conditions, W_in, Wc, We1, We2, Wx, Wh, Wout,
           combined_mask, edge_index):
    f32 = jnp.float32
    mask2 = combined_mask.reshape(N_NODES, 1).astype(jnp.int32)
    src = edge_index[0].astype(jnp.int32)
    dst = edge_index[1].astype(jnp.int32)
    xh = jnp.concatenate([pos, h], axis=1)
    wz = jnp.concatenate([jnp.zeros((3, HID), f32), W_in[:13]], axis=0)
    wt = W_in[13:14]
    we1a = We1[:HID]
    we1b = We1[HID:2 * HID]
    wd = We1[2 * HID:2 * HID + 1]
    wht = Wh[:HID]
    whb0 = Wh[HID:HID + 128]
    whb1 = Wh[HID + 128:]
    wout_pad = jnp.concatenate([jnp.zeros((HID, 3), f32), Wout], axis=1)

    mean, sc = _t1a(mask2, eps, t)
    ta, tb, pos4, hemb, epsc = _t1b(mask2, xh, eps, t, conditions, Wc, wz, wt,
                                    we1a, we1b, mean, sc)
    asp, bdp, geo = _s1(ta, tb, pos4.reshape(4 * N_NODES), src, dst)
    m2a, m2b, rlx = _t2(asp, bdp, geo, wd, We2, Wx)
    agga, aggb, dpa, dpb = _s2(m2a, m2b, rlx, dst)
    dmean = _t3a(mask2, dpa, dpb, sc)
    err = _t3b(mask2, hemb, agga, aggb, dpa, dpb, epsc, dmean, sc, wht, whb0,
               whb1, wout_pad)
    return err.reshape(NSEG)


# speedup vs baseline: 3.9459x; 1.1830x over previous
import dataclasses
import functools

import jax
import jax.numpy as jnp
from jax import lax
from jax.experimental import pallas as pl
from jax.experimental.pallas import tpu as pltpu
from jax.experimental.pallas import tpu_sc as plsc

N_NODES = 10000
N_EDGES = 160000
NSEG = 256
HID = 256
TSTEPS = 1000.0

NB = 2000
EB = 1280
GB = 128
SB = 128
TW = 256


def _silu(x):
    return x * lax.logistic(x)


def _onehot(m, rows):
    return (m == lax.broadcasted_iota(jnp.int32, (rows, NSEG), 1)).astype(
        jnp.float32)


def _t1a_body(mask_ref, eps_ref, t_ref, mean_ref, sc_ref):
    o = _onehot(mask_ref[...], N_NODES)
    ones = jnp.ones((N_NODES, 1), jnp.float32)
    cdims = (((0,), (0,)), ((), ()))
    counts = jnp.maximum(lax.dot_general(o, ones, cdims), 1.0)
    sums = lax.dot_general(o, eps_ref[...], cdims)
    mean_ref[...] = sums / counts
    t = t_ref[...]
    gamma_t = -7.0 + 13.0 * t
    gamma_s = -7.0 + 13.0 * (t - 1.0 / TSTEPS)
    alpha = jnp.sqrt(lax.logistic(-gamma_t))
    sigma = jnp.sqrt(lax.logistic(gamma_t))
    snr = 1.0 - jnp.exp(gamma_t - gamma_s)
    sc_ref[...] = jnp.concatenate(
        [counts, alpha, sigma, snr, jnp.zeros((NSEG, 4), jnp.float32)], axis=1)


def _t1a(mask2, eps, t):
    return pl.pallas_call(
        _t1a_body,
        out_shape=[
            jax.ShapeDtypeStruct((NSEG, 16), jnp.float32),
            jax.ShapeDtypeStruct((NSEG, 8), jnp.float32),
        ],
    )(mask2, eps, t)


def _t1b_body(mask_ref, xh_ref, eps_ref, t_ref, cond_ref, wc_ref, wz_ref,
              wt_ref, we1a_ref, we1b_ref, mean_ref, sc_ref,
              ta_ref, tb_ref, pos_ref, hemb_ref, epsc_ref):
    o = _onehot(mask_ref[...], NB)
    sc = sc_ref[...]
    alpha_n = o @ sc[:, 1:2]
    sigma_n = o @ sc[:, 2:3]
    t_n = o @ t_ref[...]
    cmask3 = (lax.broadcasted_iota(jnp.int32, (1, 16), 1) < 3).astype(
        jnp.float32)
    mean_n = (o @ mean_ref[...]) * cmask3
    eps_c = eps_ref[...] - mean_n
    z16 = alpha_n * xh_ref[...] + sigma_n * eps_c
    cond_n = o @ (cond_ref[...] @ wc_ref[...])
    h_emb = _silu(z16 @ wz_ref[...] + t_n @ wt_ref[...] + cond_n)
    cmask4 = (lax.broadcasted_iota(jnp.int32, (1, 4), 1) < 3).astype(
        jnp.float32)
    ta_ref[...] = h_emb @ we1a_ref[...]
    tb_ref[...] = h_emb @ we1b_ref[...]
    pos_ref[...] = z16[:, 0:4] * cmask4
    hemb_ref[...] = h_emb
    epsc_ref[...] = eps_c


def _t1b(mask2, xh, eps, t, conditions, wc, wz, wt, we1a, we1b, mean, sc):
    nblk = N_NODES // NB
    full = lambda r, c: pl.BlockSpec((r, c), lambda i: (0, 0))
    blk = lambda c: pl.BlockSpec((NB, c), lambda i: (i, 0))
    return pl.pallas_call(
        _t1b_body,
        grid=(nblk,),
        in_specs=[
            blk(1), blk(16), blk(16),
            full(NSEG, 1), full(NSEG, 1), full(1, HID),
            full(16, HID), full(1, HID), full(HID, HID), full(HID, HID),
            full(NSEG, 16), full(NSEG, 8),
        ],
        out_specs=[blk(TW), blk(TW), blk(4), blk(HID), blk(16)],
        out_shape=[
            jax.ShapeDtypeStruct((N_NODES, TW), jnp.float32),
            jax.ShapeDtypeStruct((N_NODES, TW), jnp.float32),
            jax.ShapeDtypeStruct((N_NODES, 4), jnp.float32),
            jax.ShapeDtypeStruct((N_NODES, HID), jnp.float32),
            jax.ShapeDtypeStruct((N_NODES, 16), jnp.float32),
        ],
    )(mask2, xh, eps, t, conditions, wc, wz, wt, we1a, we1b, mean, sc)


def _s1(ta, tb, pos4, src, dst):
    mesh = plsc.VectorSubcoreMesh(core_axis_name="c", subcore_axis_name="s")
    cp = pltpu.CompilerParams()
    if "needs_layout_passes" in pltpu.CompilerParams.__dataclass_fields__:
        cp = dataclasses.replace(cp, needs_layout_passes=False)
    nblk = N_EDGES // GB
    nit = pl.cdiv(nblk, 32)

    @functools.partial(
        pl.kernel, mesh=mesh, compiler_params=cp,
        out_type=[
            jax.ShapeDtypeStruct((N_EDGES, TW), jnp.float32),
            jax.ShapeDtypeStruct((N_EDGES, TW), jnp.float32),
            jax.ShapeDtypeStruct((8, N_EDGES), jnp.float32),
        ],
        scratch_types=[
            pltpu.VMEM((GB,), jnp.int32), pltpu.VMEM((GB,), jnp.int32),
            pltpu.VMEM((GB, TW), jnp.float32), pltpu.VMEM((GB, TW), jnp.float32),
            pltpu.VMEM((4 * N_NODES,), jnp.float32),
            pltpu.VMEM((8, GB), jnp.float32),
            pltpu.SemaphoreType.DMA, pltpu.SemaphoreType.DMA,
            pltpu.SemaphoreType.DMA, pltpu.SemaphoreType.DMA,
            pltpu.SemaphoreType.DMA, pltpu.SemaphoreType.DMA,
        ])
    def k(ta_hbm, tb_hbm, pos_hbm, src_hbm, dst_hbm, asp_hbm, bdp_hbm,
          geo_hbm, si, di, ra, rb, posv, gbuf, sema, semb, sia, sib, swa,
          swb):
        wid = lax.axis_index("s") * 2 + lax.axis_index("c")
        pltpu.sync_copy(pos_hbm, posv)
        zero16 = jnp.zeros((16,), jnp.float32)

        @pl.loop(0, 8)
        def _(g):
            gbuf[3, pl.ds(g * 16, 16)] = zero16
            gbuf[4, pl.ds(g * 16, 16)] = zero16
            gbuf[5, pl.ds(g * 16, 16)] = zero16
            gbuf[6, pl.ds(g * 16, 16)] = zero16
            gbuf[7, pl.ds(g * 16, 16)] = zero16

        @pl.loop(0, nit)
        def _(it):
            blk = wid + it * 32

            @pl.when(blk < nblk)
            def _():
                base = blk * GB
                pltpu.sync_copy(src_hbm.at[pl.ds(base, GB)], si)
                pltpu.sync_copy(dst_hbm.at[pl.ds(base, GB)], di)
                ca = pltpu.async_copy(ta_hbm.at[si], ra, sema)
                cb = pltpu.async_copy(tb_hbm.at[di], rb, semb)

                @pl.loop(0, GB // 16)
                def _(g):
                    sidx = si[pl.ds(g * 16, 16)] * 4
                    didx = di[pl.ds(g * 16, 16)] * 4
                    for kc in range(3):
                        ps = plsc.load_gather(posv, [sidx + kc])
                        pd = plsc.load_gather(posv, [didx + kc])
                        gbuf[kc, pl.ds(g * 16, 16)] = ps - pd

                ca.wait()
                cb.wait()
                pltpu.sync_copy(ra, asp_hbm.at[pl.ds(base, GB)])
                pltpu.sync_copy(rb, bdp_hbm.at[pl.ds(base, GB)])
                pltpu.sync_copy(gbuf, geo_hbm.at[:, pl.ds(base, GB)])

    return k(ta, tb, pos4, src, dst)


def _t2_body(a_ref, b_ref, g_ref, wd_ref, we2_ref, wx_ref, m2a_ref, m2b_ref,
             rlx_ref):
    a = a_ref[...]
    b = b_ref[...]
    eye8 = (lax.broadcasted_iota(jnp.int32, (8, 8), 0)
            == lax.broadcasted_iota(jnp.int32, (8, 8), 1)).astype(jnp.float32)
    rel = lax.dot_general(g_ref[...], eye8, (((0,), (0,)), ((), ())))
    d2 = jnp.sum(rel * rel, axis=1, keepdims=True)
    m1 = _silu(a + b + d2 @ wd_ref[...])
    m2 = _silu(m1 @ we2_ref[...])
    wx = m2 @ wx_ref[...]
    m2a_ref[...] = m2[:, :128]
    m2b_ref[...] = m2[:, 128:]
    pad128 = (lax.broadcasted_iota(jnp.int32, (8, 128), 0)
              == lax.broadcasted_iota(jnp.int32, (8, 128), 1)).astype(
                  jnp.float32)
    rlx_ref[...] = (rel * wx) @ pad128


def _t2(asp, bdp, geo, wd, we2, wx):
    nblk = N_EDGES // EB
    full = lambda r, c: pl.BlockSpec((r, c), lambda i: (0, 0))
    blk = lambda c: pl.BlockSpec((EB, c), lambda i: (i, 0))
    return pl.pallas_call(
        _t2_body,
        grid=(nblk,),
        in_specs=[blk(TW), blk(TW), pl.BlockSpec((8, EB), lambda i: (0, i)),
                  full(1, HID), full(HID, HID), full(HID, 1)],
        out_specs=[blk(128), blk(128), blk(128)],
        out_shape=[
            jax.ShapeDtypeStruct((N_EDGES, 128), jnp.float32),
            jax.ShapeDtypeStruct((N_EDGES, 128), jnp.float32),
            jax.ShapeDtypeStruct((N_EDGES, 128), jnp.float32),
        ],
    )(asp, bdp, geo, wd, we2, wx)


def _s2(m2a, m2b, rlx128, dst):
    mesh = plsc.VectorSubcoreMesh(core_axis_name="c", subcore_axis_name="s")
    cp = pltpu.CompilerParams()
    if "needs_layout_passes" in pltpu.CompilerParams.__dataclass_fields__:
        cp = dataclasses.replace(cp, needs_layout_passes=False)
    nblk = N_EDGES // SB
    nblk2 = nblk // 2
    ROWS = 632
    LAST = N_NODES - 15 * ROWS

    @functools.partial(
        pl.kernel, mesh=mesh, compiler_params=cp,
        out_type=[
            jax.ShapeDtypeStruct((N_NODES, 128), jnp.float32),
            jax.ShapeDtypeStruct((N_NODES, 128), jnp.float32),
            jax.ShapeDtypeStruct((N_NODES, 128), jnp.float32),
            jax.ShapeDtypeStruct((N_NODES, 128), jnp.float32),
        ],
        scratch_types=[
            pltpu.VMEM_SHARED((N_NODES, 128), jnp.float32),
            pltpu.VMEM((8, 128), jnp.float32),
            pltpu.VMEM((SB, 128), jnp.float32),
            pltpu.VMEM((SB, 128), jnp.float32),
            pltpu.VMEM((SB,), jnp.int32),
            pltpu.VMEM((SB,), jnp.int32),
            pltpu.SemaphoreType.DMA, pltpu.SemaphoreType.DMA,
            pltpu.SemaphoreType.DMA, pltpu.SemaphoreType.DMA,
            pltpu.SemaphoreType.DMA, pltpu.SemaphoreType.DMA,
        ])
    def k(m2a_hbm, m2b_hbm, rlx_hbm, dst_hbm, agga_hbm, aggb_hbm, dpa_hbm,
          dpb_hbm, acc, zb, mb0, mb1, ix0, ix1, si0, sm0, si1, sm1, ss0, ss1):
        c = lax.axis_index("c")
        s = lax.axis_index("s")
        off = s * ROWS
        nz = jnp.where(s == 15, LAST // 8, ROWS // 8)
        zero16 = jnp.zeros((16,), jnp.float32)

        @pl.loop(0, 8)
        def _(i):
            @pl.loop(0, 8)
            def _(j):
                zb[i, pl.ds(j * 16, 16)] = zero16

        def zero_own_rows():
            @pl.loop(0, ROWS // 8)
            def _(i):
                @pl.when(i < nz)
                def _():
                    pltpu.sync_copy(zb, acc.at[pl.ds(off + i * 8, 8)])

        def copy_out(dst_full):
            @pl.when(s < 15)
            def _():
                pltpu.sync_copy(acc.at[pl.ds(off, ROWS)],
                                dst_full.at[pl.ds(off, ROWS)])

            @pl.when(s == 15)
            def _():
                pltpu.sync_copy(acc.at[pl.ds(off, LAST)],
                                dst_full.at[pl.ds(off, LAST)])

        def wait_load(ixb, mub, semi, semm):
            pltpu.make_async_copy(dst_hbm.at[pl.ds(0, SB)], ixb, semi).wait()
            pltpu.make_async_copy(m2a_hbm.at[pl.ds(0, SB)], mub, semm).wait()

        def scat(ixb, mub, sems):
            pltpu.async_copy(mub, acc.at[ixb], sems, add=True)

        def wait_scat(ixb, mub, sems):
            pltpu.make_async_copy(mub, acc.at[ixb], sems).wait()

        def pipelined_phase(nb, load_fn):
            nj = pl.cdiv(nb, 16)

            def valid(j):
                return j * 16 + s < nb

            @pl.when(valid(0))
            def _():
                load_fn(0, ix0, mb0, si0, sm0)

            @pl.loop(0, pl.cdiv(nj, 2))
            def _(it):
                j0 = 2 * it
                j1 = 2 * it + 1
                j2 = 2 * it + 2

                @pl.when(jnp.logical_and(valid(j1), j1 >= 3))
                def _():
                    wait_scat(ix1, mb1, ss1)

                @pl.when(valid(j1))
                def _():
                    load_fn(j1, ix1, mb1, si1, sm1)

                @pl.when(valid(j0))
                def _():
                    wait_load(ix0, mb0, si0, sm0)
                    scat(ix0, mb0, ss0)

                @pl.when(valid(j2))
                def _():
                    wait_scat(ix0, mb0, ss0)
                    load_fn(j2, ix0, mb0, si0, sm0)

                @pl.when(valid(j1))
                def _():
                    wait_load(ix1, mb1, si1, sm1)
                    scat(ix1, mb1, ss1)

            @pl.when(valid(0))
            def _():
                wait_scat(ix0, mb0, ss0)

            @pl.when(valid(1))
            def _():
                wait_scat(ix1, mb1, ss1)

        def load1(j, ixb, mub, semi, semm):
            base = (j * 16 + s) * SB
            pltpu.async_copy(dst_hbm.at[pl.ds(base, SB)], ixb, semi)

            @pl.when(c == 0)
            def _():
                pltpu.async_copy(m2a_hbm.at[pl.ds(base, SB)], mub, semm)

            @pl.when(c == 1)
            def _():
                pltpu.async_copy(m2b_hbm.at[pl.ds(base, SB)], mub, semm)

        zero_own_rows()
        plsc.subcore_barrier()
        pipelined_phase(nblk, load1)
        plsc.subcore_barrier()

        @pl.when(c == 0)
        def _():
            copy_out(agga_hbm)

        @pl.when(c == 1)
        def _():
            copy_out(aggb_hbm)

        def load2(j, ixb, mub, semi, semm):
            base = (c * nblk2 + j * 16 + s) * SB
            pltpu.async_copy(dst_hbm.at[pl.ds(base, SB)], ixb, semi)
            pltpu.async_copy(rlx_hbm.at[pl.ds(base, SB)], mub, semm)

        zero_own_rows()
        plsc.subcore_barrier()
        pipelined_phase(nblk2, load2)
        plsc.subcore_barrier()

        @pl.when(c == 0)
        def _():
            copy_out(dpa_hbm)

        @pl.when(c == 1)
        def _():
            copy_out(dpb_hbm)

    return k(m2a, m2b, rlx128, dst)


def _t3a_body(mask_ref, dpa_ref, dpb_ref, sc_ref, out_ref):
    o = _onehot(mask_ref[...], N_NODES)
    dpos = dpa_ref[:, :16] + dpb_ref[:, :16]
    sums = lax.dot_general(o, dpos, (((0,), (0,)), ((), ())))
    out_ref[...] = sums / sc_ref[:, 0:1]


def _t3a(mask2, dpa, dpb, sc):
    return pl.pallas_call(
        _t3a_body,
        out_shape=jax.ShapeDtypeStruct((NSEG, 16), jnp.float32),
    )(mask2, dpa, dpb, sc)


def _t3b_body(mask_ref, hemb_ref, agga_ref, aggb_ref, dpa_ref, dpb_ref,
              epsc_ref, dmean_ref, sc_ref, wht_ref, whb0_ref, whb1_ref,
              wout_ref, out_ref):
    i = pl.program_id(0)
    o = _onehot(mask_ref[...], NB)
    h_new = _silu(hemb_ref[...] @ wht_ref[...] + agga_ref[...] @ whb0_ref[...]
                  + aggb_ref[...] @ whb1_ref[...])
    dpos = dpa_ref[:, :16] + dpb_ref[:, :16]
    net16 = (dpos - o @ dmean_ref[...]) + h_new @ wout_ref[...]
    diff = epsc_ref[...] - net16
    err = jnp.sum(diff * diff, axis=1, keepdims=True)
    part = lax.dot_general(o, err, (((0,), (0,)), ((), ())))

    @pl.when(i == 0)
    def _():
        out_ref[...] = part

    @pl.when(i > 0)
    def _():
        out_ref[...] += part

    @pl.when(i == N_NODES // NB - 1)
    def _():
        out_ref[...] *= sc_ref[:, 3:4]


def _t3b(mask2, hemb, agga, aggb, dpa, dpb, epsc, dmean, sc, wht, whb0, whb1,
         wout_pad):
    nblk = N_NODES // NB
    full = lambda r, c: pl.BlockSpec((r, c), lambda i: (0, 0))
    blk = lambda c: pl.BlockSpec((NB, c), lambda i: (i, 0))
    return pl.pallas_call(
        _t3b_body,
        grid=(nblk,),
        in_specs=[
            blk(1), blk(HID), blk(128), blk(128), blk(128), blk(128), blk(16),
            full(NSEG, 16), full(NSEG, 8),
            full(HID, HID), full(128, HID), full(128, HID), full(HID, 16),
        ],
        out_specs=pl.BlockSpec((NSEG, 1), lambda i: (0, 0)),
        out_shape=jax.ShapeDtypeStruct((NSEG, 1), jnp.float32),
    )(mask2, hemb, agga, aggb, dpa, dpb, epsc, dmean, sc, wht, whb0, whb1,
      wout_pad)


def kernel(pos, h, eps, t, conditions, W_in, Wc, We1, We2, Wx, Wh, Wout,
           combined_mask, edge_index):
    f32 = jnp.float32
    mask2 = combined_mask.reshape(N_NODES, 1).astype(jnp.int32)
    src = edge_index[0].astype(jnp.int32)
    dst = edge_index[1].astype(jnp.int32)
    xh = jnp.concatenate([pos, h], axis=1)
    wz = jnp.concatenate([jnp.zeros((3, HID), f32), W_in[:13]], axis=0)
    wt = W_in[13:14]
    we1a = We1[:HID]
    we1b = We1[HID:2 * HID]
    wd = We1[2 * HID:2 * HID + 1]
    wht = Wh[:HID]
    whb0 = Wh[HID:HID + 128]
    whb1 = Wh[HID + 128:]
    wout_pad = jnp.concatenate([jnp.zeros((HID, 3), f32), Wout], axis=1)

    mean, sc = _t1a(mask2, eps, t)
    ta, tb, pos4, hemb, epsc = _t1b(mask2, xh, eps, t, conditions, Wc, wz, wt,
                                    we1a, we1b, mean, sc)
    asp, bdp, geo = _s1(ta, tb, pos4.reshape(4 * N_NODES), src, dst)
    m2a, m2b, rlx = _t2(asp, bdp, geo, wd, We2, Wx)
    agga, aggb, dpa, dpb = _s2(m2a, m2b, rlx, dst)
    dmean = _t3a(mask2, dpa, dpb, sc)
    err = _t3b(mask2, hemb, agga, aggb, dpa, dpb, epsc, dmean, sc, wht, whb0,
               whb1, wout_pad)
    return err.reshape(NSEG)


# speedup vs baseline: 3.9501x; 1.0011x over previous
import dataclasses
import functools

import jax
import jax.numpy as jnp
from jax import lax
from jax.experimental import pallas as pl
from jax.experimental.pallas import tpu as pltpu
from jax.experimental.pallas import tpu_sc as plsc

N_NODES = 10000
N_EDGES = 160000
NSEG = 256
HID = 256
TSTEPS = 1000.0

NB = 2000
EB = 1280
GB = 128
SB = 128
TW = 256


def _silu(x):
    return x * lax.logistic(x)


def _onehot(m, rows):
    return (m == lax.broadcasted_iota(jnp.int32, (rows, NSEG), 1)).astype(
        jnp.float32)


def _t1a_body(mask_ref, eps_ref, t_ref, mean_ref, sc_ref):
    o = _onehot(mask_ref[...], N_NODES)
    ones = jnp.ones((N_NODES, 1), jnp.float32)
    cdims = (((0,), (0,)), ((), ()))
    counts = jnp.maximum(lax.dot_general(o, ones, cdims), 1.0)
    sums = lax.dot_general(o, eps_ref[...], cdims)
    mean_ref[...] = sums / counts
    t = t_ref[...]
    gamma_t = -7.0 + 13.0 * t
    gamma_s = -7.0 + 13.0 * (t - 1.0 / TSTEPS)
    alpha = jnp.sqrt(lax.logistic(-gamma_t))
    sigma = jnp.sqrt(lax.logistic(gamma_t))
    snr = 1.0 - jnp.exp(gamma_t - gamma_s)
    sc_ref[...] = jnp.concatenate(
        [counts, alpha, sigma, snr, jnp.zeros((NSEG, 4), jnp.float32)], axis=1)


def _t1a(mask2, eps, t):
    return pl.pallas_call(
        _t1a_body,
        out_shape=[
            jax.ShapeDtypeStruct((NSEG, 16), jnp.float32),
            jax.ShapeDtypeStruct((NSEG, 8), jnp.float32),
        ],
    )(mask2, eps, t)


def _t1b_body(mask_ref, xh_ref, eps_ref, t_ref, cond_ref, wc_ref, wz_ref,
              wt_ref, we1a_ref, we1b_ref, mean_ref, sc_ref,
              ta_ref, tb_ref, pos_ref, hemb_ref, epsc_ref):
    o = _onehot(mask_ref[...], NB)
    sc = sc_ref[...]
    alpha_n = o @ sc[:, 1:2]
    sigma_n = o @ sc[:, 2:3]
    t_n = o @ t_ref[...]
    cmask3 = (lax.broadcasted_iota(jnp.int32, (1, 16), 1) < 3).astype(
        jnp.float32)
    mean_n = (o @ mean_ref[...]) * cmask3
    eps_c = eps_ref[...] - mean_n
    z16 = alpha_n * xh_ref[...] + sigma_n * eps_c
    cond_n = o @ (cond_ref[...] @ wc_ref[...])
    h_emb = _silu(z16 @ wz_ref[...] + t_n @ wt_ref[...] + cond_n)
    cmask4 = (lax.broadcasted_iota(jnp.int32, (1, 4), 1) < 3).astype(
        jnp.float32)
    hb = h_emb.astype(jnp.bfloat16)
    ta_ref[...] = jnp.dot(hb, we1a_ref[...].astype(jnp.bfloat16),
                          preferred_element_type=jnp.float32)
    tb_ref[...] = jnp.dot(hb, we1b_ref[...].astype(jnp.bfloat16),
                          preferred_element_type=jnp.float32)
    pos_ref[...] = z16[:, 0:4] * cmask4
    hemb_ref[...] = h_emb
    epsc_ref[...] = eps_c


def _t1b(mask2, xh, eps, t, conditions, wc, wz, wt, we1a, we1b, mean, sc):
    nblk = N_NODES // NB
    full = lambda r, c: pl.BlockSpec((r, c), lambda i: (0, 0))
    blk = lambda c: pl.BlockSpec((NB, c), lambda i: (i, 0))
    return pl.pallas_call(
        _t1b_body,
        grid=(nblk,),
        in_specs=[
            blk(1), blk(16), blk(16),
            full(NSEG, 1), full(NSEG, 1), full(1, HID),
            full(16, HID), full(1, HID), full(HID, HID), full(HID, HID),
            full(NSEG, 16), full(NSEG, 8),
        ],
        out_specs=[blk(TW), blk(TW), blk(4), blk(HID), blk(16)],
        out_shape=[
            jax.ShapeDtypeStruct((N_NODES, TW), jnp.float32),
            jax.ShapeDtypeStruct((N_NODES, TW), jnp.float32),
            jax.ShapeDtypeStruct((N_NODES, 4), jnp.float32),
            jax.ShapeDtypeStruct((N_NODES, HID), jnp.float32),
            jax.ShapeDtypeStruct((N_NODES, 16), jnp.float32),
        ],
    )(mask2, xh, eps, t, conditions, wc, wz, wt, we1a, we1b, mean, sc)


def _s1(ta, tb, pos4, src, dst):
    mesh = plsc.VectorSubcoreMesh(core_axis_name="c", subcore_axis_name="s")
    cp = pltpu.CompilerParams()
    if "needs_layout_passes" in pltpu.CompilerParams.__dataclass_fields__:
        cp = dataclasses.replace(cp, needs_layout_passes=False)
    nblk = N_EDGES // GB
    nit = pl.cdiv(nblk, 32)

    @functools.partial(
        pl.kernel, mesh=mesh, compiler_params=cp,
        out_type=[
            jax.ShapeDtypeStruct((N_EDGES, TW), jnp.float32),
            jax.ShapeDtypeStruct((N_EDGES, TW), jnp.float32),
            jax.ShapeDtypeStruct((8, N_EDGES), jnp.float32),
        ],
        scratch_types=[
            pltpu.VMEM((GB,), jnp.int32), pltpu.VMEM((GB,), jnp.int32),
            pltpu.VMEM((GB, TW), jnp.float32), pltpu.VMEM((GB, TW), jnp.float32),
            pltpu.VMEM((4 * N_NODES,), jnp.float32),
            pltpu.VMEM((8, GB), jnp.float32),
            pltpu.SemaphoreType.DMA, pltpu.SemaphoreType.DMA,
            pltpu.SemaphoreType.DMA, pltpu.SemaphoreType.DMA,
            pltpu.SemaphoreType.DMA, pltpu.SemaphoreType.DMA,
        ])
    def k(ta_hbm, tb_hbm, pos_hbm, src_hbm, dst_hbm, asp_hbm, bdp_hbm,
          geo_hbm, si, di, ra, rb, posv, gbuf, sema, semb, sia, sib, swa,
          swb):
        wid = lax.axis_index("s") * 2 + lax.axis_index("c")
        pltpu.sync_copy(pos_hbm, posv)
        zero16 = jnp.zeros((16,), jnp.float32)

        @pl.loop(0, 8)
        def _(g):
            gbuf[3, pl.ds(g * 16, 16)] = zero16
            gbuf[4, pl.ds(g * 16, 16)] = zero16
            gbuf[5, pl.ds(g * 16, 16)] = zero16
            gbuf[6, pl.ds(g * 16, 16)] = zero16
            gbuf[7, pl.ds(g * 16, 16)] = zero16

        @pl.loop(0, nit)
        def _(it):
            blk = wid + it * 32

            @pl.when(blk < nblk)
            def _():
                base = blk * GB
                pltpu.sync_copy(src_hbm.at[pl.ds(base, GB)], si)
                pltpu.sync_copy(dst_hbm.at[pl.ds(base, GB)], di)
                ca = pltpu.async_copy(ta_hbm.at[si], ra, sema)
                cb = pltpu.async_copy(tb_hbm.at[di], rb, semb)

                @pl.loop(0, GB // 16)
                def _(g):
                    sidx = si[pl.ds(g * 16, 16)] * 4
                    didx = di[pl.ds(g * 16, 16)] * 4
                    for kc in range(3):
                        ps = plsc.load_gather(posv, [sidx + kc])
                        pd = plsc.load_gather(posv, [didx + kc])
                        gbuf[kc, pl.ds(g * 16, 16)] = ps - pd

                ca.wait()
                cb.wait()
                pltpu.sync_copy(ra, asp_hbm.at[pl.ds(base, GB)])
                pltpu.sync_copy(rb, bdp_hbm.at[pl.ds(base, GB)])
                pltpu.sync_copy(gbuf, geo_hbm.at[:, pl.ds(base, GB)])

    return k(ta, tb, pos4, src, dst)


def _t2_body(a_ref, b_ref, g_ref, wd_ref, we2_ref, wx_ref, m2a_ref, m2b_ref,
             rlx_ref):
    a = a_ref[...]
    b = b_ref[...]
    eye8 = (lax.broadcasted_iota(jnp.int32, (8, 8), 0)
            == lax.broadcasted_iota(jnp.int32, (8, 8), 1)).astype(jnp.float32)
    rel = lax.dot_general(g_ref[...], eye8, (((0,), (0,)), ((), ())))
    d2 = jnp.sum(rel * rel, axis=1, keepdims=True)
    m1 = _silu(a + b + d2 @ wd_ref[...])
    m2 = _silu(jnp.dot(m1.astype(jnp.bfloat16),
                       we2_ref[...].astype(jnp.bfloat16),
                       preferred_element_type=jnp.float32))
    wx = m2 @ wx_ref[...]
    m2a_ref[...] = m2[:, :128]
    m2b_ref[...] = m2[:, 128:]
    pad128 = (lax.broadcasted_iota(jnp.int32, (8, 128), 0)
              == lax.broadcasted_iota(jnp.int32, (8, 128), 1)).astype(
                  jnp.float32)
    rlx_ref[...] = (rel * wx) @ pad128


def _t2(asp, bdp, geo, wd, we2, wx):
    nblk = N_EDGES // EB
    full = lambda r, c: pl.BlockSpec((r, c), lambda i: (0, 0))
    blk = lambda c: pl.BlockSpec((EB, c), lambda i: (i, 0))
    return pl.pallas_call(
        _t2_body,
        grid=(nblk,),
        in_specs=[blk(TW), blk(TW), pl.BlockSpec((8, EB), lambda i: (0, i)),
                  full(1, HID), full(HID, HID), full(HID, 1)],
        out_specs=[blk(128), blk(128), blk(128)],
        out_shape=[
            jax.ShapeDtypeStruct((N_EDGES, 128), jnp.float32),
            jax.ShapeDtypeStruct((N_EDGES, 128), jnp.float32),
            jax.ShapeDtypeStruct((N_EDGES, 128), jnp.float32),
        ],
    )(asp, bdp, geo, wd, we2, wx)


def _s2(m2a, m2b, rlx128, dst):
    mesh = plsc.VectorSubcoreMesh(core_axis_name="c", subcore_axis_name="s")
    cp = pltpu.CompilerParams()
    if "needs_layout_passes" in pltpu.CompilerParams.__dataclass_fields__:
        cp = dataclasses.replace(cp, needs_layout_passes=False)
    nblk = N_EDGES // SB
    nblk2 = nblk // 2
    ROWS = 632
    LAST = N_NODES - 15 * ROWS

    @functools.partial(
        pl.kernel, mesh=mesh, compiler_params=cp,
        out_type=[
            jax.ShapeDtypeStruct((N_NODES, 128), jnp.float32),
            jax.ShapeDtypeStruct((N_NODES, 128), jnp.float32),
            jax.ShapeDtypeStruct((N_NODES, 128), jnp.float32),
            jax.ShapeDtypeStruct((N_NODES, 128), jnp.float32),
        ],
        scratch_types=[
            pltpu.VMEM_SHARED((N_NODES, 128), jnp.float32),
            pltpu.VMEM((8, 128), jnp.float32),
            pltpu.VMEM((SB, 128), jnp.float32),
            pltpu.VMEM((SB, 128), jnp.float32),
            pltpu.VMEM((SB,), jnp.int32),
            pltpu.VMEM((SB,), jnp.int32),
            pltpu.SemaphoreType.DMA, pltpu.SemaphoreType.DMA,
            pltpu.SemaphoreType.DMA, pltpu.SemaphoreType.DMA,
            pltpu.SemaphoreType.DMA, pltpu.SemaphoreType.DMA,
        ])
    def k(m2a_hbm, m2b_hbm, rlx_hbm, dst_hbm, agga_hbm, aggb_hbm, dpa_hbm,
          dpb_hbm, acc, zb, mb0, mb1, ix0, ix1, si0, sm0, si1, sm1, ss0, ss1):
        c = lax.axis_index("c")
        s = lax.axis_index("s")
        off = s * ROWS
        nz = jnp.where(s == 15, LAST // 8, ROWS // 8)
        zero16 = jnp.zeros((16,), jnp.float32)

        @pl.loop(0, 8)
        def _(i):
            @pl.loop(0, 8)
            def _(j):
                zb[i, pl.ds(j * 16, 16)] = zero16

        def zero_own_rows():
            @pl.loop(0, ROWS // 8)
            def _(i):
                @pl.when(i < nz)
                def _():
                    pltpu.sync_copy(zb, acc.at[pl.ds(off + i * 8, 8)])

        def copy_out(dst_full):
            @pl.when(s < 15)
            def _():
                pltpu.sync_copy(acc.at[pl.ds(off, ROWS)],
                                dst_full.at[pl.ds(off, ROWS)])

            @pl.when(s == 15)
            def _():
                pltpu.sync_copy(acc.at[pl.ds(off, LAST)],
                                dst_full.at[pl.ds(off, LAST)])

        def wait_load(ixb, mub, semi, semm):
            pltpu.make_async_copy(dst_hbm.at[pl.ds(0, SB)], ixb, semi).wait()
            pltpu.make_async_copy(m2a_hbm.at[pl.ds(0, SB)], mub, semm).wait()

        def scat(ixb, mub, sems):
            pltpu.async_copy(mub, acc.at[ixb], sems, add=True)

        def wait_scat(ixb, mub, sems):
            pltpu.make_async_copy(mub, acc.at[ixb], sems).wait()

        def pipelined_phase(nb, load_fn):
            nj = pl.cdiv(nb, 16)

            def valid(j):
                return j * 16 + s < nb

            @pl.when(valid(0))
            def _():
                load_fn(0, ix0, mb0, si0, sm0)

            @pl.loop(0, pl.cdiv(nj, 2))
            def _(it):
                j0 = 2 * it
                j1 = 2 * it + 1
                j2 = 2 * it + 2

                @pl.when(jnp.logical_and(valid(j1), j1 >= 3))
                def _():
                    wait_scat(ix1, mb1, ss1)

                @pl.when(valid(j1))
                def _():
                    load_fn(j1, ix1, mb1, si1, sm1)

                @pl.when(valid(j0))
                def _():
                    wait_load(ix0, mb0, si0, sm0)
                    scat(ix0, mb0, ss0)

                @pl.when(valid(j2))
                def _():
                    wait_scat(ix0, mb0, ss0)
                    load_fn(j2, ix0, mb0, si0, sm0)

                @pl.when(valid(j1))
                def _():
                    wait_load(ix1, mb1, si1, sm1)
                    scat(ix1, mb1, ss1)

            @pl.when(valid(0))
            def _():
                wait_scat(ix0, mb0, ss0)

            @pl.when(valid(1))
            def _():
                wait_scat(ix1, mb1, ss1)

        def load1(j, ixb, mub, semi, semm):
            base = (j * 16 + s) * SB
            pltpu.async_copy(dst_hbm.at[pl.ds(base, SB)], ixb, semi)

            @pl.when(c == 0)
            def _():
                pltpu.async_copy(m2a_hbm.at[pl.ds(base, SB)], mub, semm)

            @pl.when(c == 1)
            def _():
                pltpu.async_copy(m2b_hbm.at[pl.ds(base, SB)], mub, semm)

        zero_own_rows()
        plsc.subcore_barrier()
        pipelined_phase(nblk, load1)
        plsc.subcore_barrier()

        @pl.when(c == 0)
        def _():
            copy_out(agga_hbm)

        @pl.when(c == 1)
        def _():
            copy_out(aggb_hbm)

        def load2(j, ixb, mub, semi, semm):
            base = (c * nblk2 + j * 16 + s) * SB
            pltpu.async_copy(dst_hbm.at[pl.ds(base, SB)], ixb, semi)
            pltpu.async_copy(rlx_hbm.at[pl.ds(base, SB)], mub, semm)

        zero_own_rows()
        plsc.subcore_barrier()
        pipelined_phase(nblk2, load2)
        plsc.subcore_barrier()

        @pl.when(c == 0)
        def _():
            copy_out(dpa_hbm)

        @pl.when(c == 1)
        def _():
            copy_out(dpb_hbm)

    return k(m2a, m2b, rlx128, dst)


def _t3a_body(mask_ref, dpa_ref, dpb_ref, sc_ref, out_ref):
    o = _onehot(mask_ref[...], N_NODES)
    dpos = dpa_ref[:, :16] + dpb_ref[:, :16]
    sums = lax.dot_general(o, dpos, (((0,), (0,)), ((), ())))
    out_ref[...] = sums / sc_ref[:, 0:1]


def _t3a(mask2, dpa, dpb, sc):
    return pl.pallas_call(
        _t3a_body,
        out_shape=jax.ShapeDtypeStruct((NSEG, 16), jnp.float32),
    )(mask2, dpa, dpb, sc)


def _t3b_body(mask_ref, hemb_ref, agga_ref, aggb_ref, dpa_ref, dpb_ref,
              epsc_ref, dmean_ref, sc_ref, wht_ref, whb0_ref, whb1_ref,
              wout_ref, out_ref):
    i = pl.program_id(0)
    o = _onehot(mask_ref[...], NB)
    h_new = _silu(
        jnp.dot(hemb_ref[...].astype(jnp.bfloat16),
                wht_ref[...].astype(jnp.bfloat16),
                preferred_element_type=jnp.float32)
        + jnp.dot(agga_ref[...].astype(jnp.bfloat16),
                  whb0_ref[...].astype(jnp.bfloat16),
                  preferred_element_type=jnp.float32)
        + jnp.dot(aggb_ref[...].astype(jnp.bfloat16),
                  whb1_ref[...].astype(jnp.bfloat16),
                  preferred_element_type=jnp.float32))
    dpos = dpa_ref[:, :16] + dpb_ref[:, :16]
    net16 = (dpos - o @ dmean_ref[...]) + h_new @ wout_ref[...]
    diff = epsc_ref[...] - net16
    err = jnp.sum(diff * diff, axis=1, keepdims=True)
    part = lax.dot_general(o, err, (((0,), (0,)), ((), ())))

    @pl.when(i == 0)
    def _():
        out_ref[...] = part

    @pl.when(i > 0)
    def _():
        out_ref[...] += part

    @pl.when(i == N_NODES // NB - 1)
    def _():
        out_ref[...] *= sc_ref[:, 3:4]


def _t3b(mask2, hemb, agga, aggb, dpa, dpb, epsc, dmean, sc, wht, whb0, whb1,
         wout_pad):
    nblk = N_NODES // NB
    full = lambda r, c: pl.BlockSpec((r, c), lambda i: (0, 0))
    blk = lambda c: pl.BlockSpec((NB, c), lambda i: (i, 0))
    return pl.pallas_call(
        _t3b_body,
        grid=(nblk,),
        in_specs=[
            blk(1), blk(HID), blk(128), blk(128), blk(128), blk(128), blk(16),
            full(NSEG, 16), full(NSEG, 8),
            full(HID, HID), full(128, HID), full(128, HID), full(HID, 16),
        ],
        out_specs=pl.BlockSpec((NSEG, 1), lambda i: (0, 0)),
        out_shape=jax.ShapeDtypeStruct((NSEG, 1), jnp.float32),
    )(mask2, hemb, agga, aggb, dpa, dpb, epsc, dmean, sc, wht, whb0, whb1,
      wout_pad)


def kernel(pos, h, eps, t, conditions, W_in, Wc, We1, We2, Wx, Wh, Wout,
           combined_mask, edge_index):
    f32 = jnp.float32
    mask2 = combined_mask.reshape(N_NODES, 1).astype(jnp.int32)
    src = edge_index[0].astype(jnp.int32)
    dst = edge_index[1].astype(jnp.int32)
    xh = jnp.concatenate([pos, h], axis=1)
    wz = jnp.concatenate([jnp.zeros((3, HID), f32), W_in[:13]], axis=0)
    wt = W_in[13:14]
    we1a = We1[:HID]
    we1b = We1[HID:2 * HID]
    wd = We1[2 * HID:2 * HID + 1]
    wht = Wh[:HID]
    whb0 = Wh[HID:HID + 128]
    whb1 = Wh[HID + 128:]
    wout_pad = jnp.concatenate([jnp.zeros((HID, 3), f32), Wout], axis=1)

    mean, sc = _t1a(mask2, eps, t)
    ta, tb, pos4, hemb, epsc = _t1b(mask2, xh, eps, t, conditions, Wc, wz, wt,
                                    we1a, we1b, mean, sc)
    asp, bdp, geo = _s1(ta, tb, pos4.reshape(4 * N_NODES), src, dst)
    m2a, m2b, rlx = _t2(asp, bdp, geo, wd, We2, Wx)
    agga, aggb, dpa, dpb = _s2(m2a, m2b, rlx, dst)
    dmean = _t3a(mask2, dpa, dpb, sc)
    err = _t3b(mask2, hemb, agga, aggb, dpa, dpb, epsc, dmean, sc, wht, whb0,
               whb1, wout_pad)
    return err.reshape(NSEG)


# speedup vs baseline: 4.7763x; 1.2092x over previous
import dataclasses
import functools

import jax
import jax.numpy as jnp
from jax import lax
from jax.experimental import pallas as pl
from jax.experimental.pallas import tpu as pltpu
from jax.experimental.pallas import tpu_sc as plsc

N_NODES = 10000
N_EDGES = 160000
NSEG = 256
HID = 256
TSTEPS = 1000.0

NB = 2000
EB = 1280
GB = 128
SB = 128
TW = 128


def _silu(x):
    return x * lax.logistic(x)


def _pack_pair(x):
    rows = x.shape[0]
    bits = lax.bitcast_convert_type(x, jnp.int32)
    rounded = bits + 0x8000
    lo = lax.shift_right_logical(rounded[:, :TW], 16)
    hi = jnp.bitwise_and(rounded[:, TW:], jnp.int32(-65536))
    return lax.bitcast_convert_type(jnp.bitwise_or(hi, lo), jnp.float32)


def _unpack_pair(p):
    w = lax.bitcast_convert_type(p, jnp.int32)
    lo = lax.bitcast_convert_type(lax.shift_left(w, 16), jnp.float32)
    hi = lax.bitcast_convert_type(
        jnp.bitwise_and(w, jnp.int32(-65536)), jnp.float32)
    return jnp.concatenate([lo, hi], axis=1)


def _onehot(m, rows):
    return (m == lax.broadcasted_iota(jnp.int32, (rows, NSEG), 1)).astype(
        jnp.float32)


def _t1a_body(mask_ref, eps_ref, t_ref, mean_ref, sc_ref):
    o = _onehot(mask_ref[...], N_NODES)
    ones = jnp.ones((N_NODES, 1), jnp.float32)
    cdims = (((0,), (0,)), ((), ()))
    counts = jnp.maximum(lax.dot_general(o, ones, cdims), 1.0)
    sums = lax.dot_general(o, eps_ref[...], cdims)
    mean_ref[...] = sums / counts
    t = t_ref[...]
    gamma_t = -7.0 + 13.0 * t
    gamma_s = -7.0 + 13.0 * (t - 1.0 / TSTEPS)
    alpha = jnp.sqrt(lax.logistic(-gamma_t))
    sigma = jnp.sqrt(lax.logistic(gamma_t))
    snr = 1.0 - jnp.exp(gamma_t - gamma_s)
    sc_ref[...] = jnp.concatenate(
        [counts, alpha, sigma, snr, jnp.zeros((NSEG, 4), jnp.float32)], axis=1)


def _t1a(mask2, eps, t):
    return pl.pallas_call(
        _t1a_body,
        out_shape=[
            jax.ShapeDtypeStruct((NSEG, 16), jnp.float32),
            jax.ShapeDtypeStruct((NSEG, 8), jnp.float32),
        ],
    )(mask2, eps, t)


def _t1b_body(mask_ref, xh_ref, eps_ref, t_ref, cond_ref, wc_ref, wz_ref,
              wt_ref, we1a_ref, we1b_ref, mean_ref, sc_ref,
              ta_ref, tb_ref, pos_ref, hemb_ref, epsc_ref):
    o = _onehot(mask_ref[...], NB)
    sc = sc_ref[...]
    alpha_n = o @ sc[:, 1:2]
    sigma_n = o @ sc[:, 2:3]
    t_n = o @ t_ref[...]
    cmask3 = (lax.broadcasted_iota(jnp.int32, (1, 16), 1) < 3).astype(
        jnp.float32)
    mean_n = (o @ mean_ref[...]) * cmask3
    eps_c = eps_ref[...] - mean_n
    z16 = alpha_n * xh_ref[...] + sigma_n * eps_c
    cond_n = o @ (cond_ref[...] @ wc_ref[...])
    h_emb = _silu(z16 @ wz_ref[...] + t_n @ wt_ref[...] + cond_n)
    cmask4 = (lax.broadcasted_iota(jnp.int32, (1, 4), 1) < 3).astype(
        jnp.float32)
    hb = h_emb.astype(jnp.bfloat16)
    ta32 = jnp.dot(hb, we1a_ref[...].astype(jnp.bfloat16),
                   preferred_element_type=jnp.float32)
    tb32 = jnp.dot(hb, we1b_ref[...].astype(jnp.bfloat16),
                   preferred_element_type=jnp.float32)
    ta_ref[...] = _pack_pair(ta32)
    tb_ref[...] = _pack_pair(tb32)
    pos_ref[...] = z16[:, 0:4] * cmask4
    hemb_ref[...] = h_emb
    epsc_ref[...] = eps_c


def _t1b(mask2, xh, eps, t, conditions, wc, wz, wt, we1a, we1b, mean, sc):
    nblk = N_NODES // NB
    full = lambda r, c: pl.BlockSpec((r, c), lambda i: (0, 0))
    blk = lambda c: pl.BlockSpec((NB, c), lambda i: (i, 0))
    return pl.pallas_call(
        _t1b_body,
        grid=(nblk,),
        in_specs=[
            blk(1), blk(16), blk(16),
            full(NSEG, 1), full(NSEG, 1), full(1, HID),
            full(16, HID), full(1, HID), full(HID, HID), full(HID, HID),
            full(NSEG, 16), full(NSEG, 8),
        ],
        out_specs=[blk(TW), blk(TW), blk(4), blk(HID), blk(16)],
        out_shape=[
            jax.ShapeDtypeStruct((N_NODES, TW), jnp.float32),
            jax.ShapeDtypeStruct((N_NODES, TW), jnp.float32),
            jax.ShapeDtypeStruct((N_NODES, 4), jnp.float32),
            jax.ShapeDtypeStruct((N_NODES, HID), jnp.float32),
            jax.ShapeDtypeStruct((N_NODES, 16), jnp.float32),
        ],
    )(mask2, xh, eps, t, conditions, wc, wz, wt, we1a, we1b, mean, sc)


def _s1(ta, tb, pos4, src, dst):
    mesh = plsc.VectorSubcoreMesh(core_axis_name="c", subcore_axis_name="s")
    cp = pltpu.CompilerParams()
    if "needs_layout_passes" in pltpu.CompilerParams.__dataclass_fields__:
        cp = dataclasses.replace(cp, needs_layout_passes=False)
    nblk = N_EDGES // GB
    nit = pl.cdiv(nblk, 32)

    @functools.partial(
        pl.kernel, mesh=mesh, compiler_params=cp,
        out_type=[
            jax.ShapeDtypeStruct((N_EDGES, TW), jnp.float32),
            jax.ShapeDtypeStruct((N_EDGES, TW), jnp.float32),
            jax.ShapeDtypeStruct((8, N_EDGES), jnp.float32),
        ],
        scratch_types=[
            pltpu.VMEM((GB,), jnp.int32), pltpu.VMEM((GB,), jnp.int32),
            pltpu.VMEM((GB, TW), jnp.float32), pltpu.VMEM((GB, TW), jnp.float32),
            pltpu.VMEM((4 * N_NODES,), jnp.float32),
            pltpu.VMEM((8, GB), jnp.float32),
            pltpu.SemaphoreType.DMA, pltpu.SemaphoreType.DMA,
            pltpu.SemaphoreType.DMA, pltpu.SemaphoreType.DMA,
            pltpu.SemaphoreType.DMA, pltpu.SemaphoreType.DMA,
        ])
    def k(ta_hbm, tb_hbm, pos_hbm, src_hbm, dst_hbm, asp_hbm, bdp_hbm,
          geo_hbm, si, di, ra, rb, posv, gbuf, sema, semb, sia, sib, swa,
          swb):
        wid = lax.axis_index("s") * 2 + lax.axis_index("c")
        pltpu.sync_copy(pos_hbm, posv)
        zero16 = jnp.zeros((16,), jnp.float32)

        @pl.loop(0, 8)
        def _(g):
            gbuf[3, pl.ds(g * 16, 16)] = zero16
            gbuf[4, pl.ds(g * 16, 16)] = zero16
            gbuf[5, pl.ds(g * 16, 16)] = zero16
            gbuf[6, pl.ds(g * 16, 16)] = zero16
            gbuf[7, pl.ds(g * 16, 16)] = zero16

        @pl.loop(0, nit)
        def _(it):
            blk = wid + it * 32

            @pl.when(blk < nblk)
            def _():
                base = blk * GB
                pltpu.sync_copy(src_hbm.at[pl.ds(base, GB)], si)
                pltpu.sync_copy(dst_hbm.at[pl.ds(base, GB)], di)
                ca = pltpu.async_copy(ta_hbm.at[si], ra, sema)
                cb = pltpu.async_copy(tb_hbm.at[di], rb, semb)

                @pl.loop(0, GB // 16)
                def _(g):
                    sidx = si[pl.ds(g * 16, 16)] * 4
                    didx = di[pl.ds(g * 16, 16)] * 4
                    for kc in range(3):
                        ps = plsc.load_gather(posv, [sidx + kc])
                        pd = plsc.load_gather(posv, [didx + kc])
                        gbuf[kc, pl.ds(g * 16, 16)] = ps - pd

                ca.wait()
                cb.wait()
                pltpu.sync_copy(ra, asp_hbm.at[pl.ds(base, GB)])
                pltpu.sync_copy(rb, bdp_hbm.at[pl.ds(base, GB)])
                pltpu.sync_copy(gbuf, geo_hbm.at[:, pl.ds(base, GB)])

    return k(ta, tb, pos4, src, dst)


def _t2_body(a_ref, b_ref, g_ref, wd_ref, we2_ref, wx_ref, m2a_ref, m2b_ref,
             rlx_ref):
    a = _unpack_pair(a_ref[...])
    b = _unpack_pair(b_ref[...])
    eye8 = (lax.broadcasted_iota(jnp.int32, (8, 8), 0)
            == lax.broadcasted_iota(jnp.int32, (8, 8), 1)).astype(jnp.float32)
    rel = lax.dot_general(g_ref[...], eye8, (((0,), (0,)), ((), ())))
    d2 = jnp.sum(rel * rel, axis=1, keepdims=True)
    m1 = _silu(a + b + d2 @ wd_ref[...])
    m2 = _silu(jnp.dot(m1.astype(jnp.bfloat16),
                       we2_ref[...].astype(jnp.bfloat16),
                       preferred_element_type=jnp.float32))
    wx = m2 @ wx_ref[...]
    m2a_ref[...] = m2[:, :128]
    m2b_ref[...] = m2[:, 128:]
    pad128 = (lax.broadcasted_iota(jnp.int32, (8, 128), 0)
              == lax.broadcasted_iota(jnp.int32, (8, 128), 1)).astype(
                  jnp.float32)
    rlx_ref[...] = (rel * wx) @ pad128


def _t2(asp, bdp, geo, wd, we2, wx):
    nblk = N_EDGES // EB
    full = lambda r, c: pl.BlockSpec((r, c), lambda i: (0, 0))
    blk = lambda c: pl.BlockSpec((EB, c), lambda i: (i, 0))
    return pl.pallas_call(
        _t2_body,
        grid=(nblk,),
        in_specs=[blk(TW), blk(TW), pl.BlockSpec((8, EB), lambda i: (0, i)),
                  full(1, HID), full(HID, HID), full(HID, 1)],
        out_specs=[blk(128), blk(128), blk(128)],
        out_shape=[
            jax.ShapeDtypeStruct((N_EDGES, 128), jnp.float32),
            jax.ShapeDtypeStruct((N_EDGES, 128), jnp.float32),
            jax.ShapeDtypeStruct((N_EDGES, 128), jnp.float32),
        ],
    )(asp, bdp, geo, wd, we2, wx)


def _s2(m2a, m2b, rlx128, dst):
    mesh = plsc.VectorSubcoreMesh(core_axis_name="c", subcore_axis_name="s")
    cp = pltpu.CompilerParams()
    if "needs_layout_passes" in pltpu.CompilerParams.__dataclass_fields__:
        cp = dataclasses.replace(cp, needs_layout_passes=False)
    nblk = N_EDGES // SB
    nblk2 = nblk // 2
    ROWS = 632
    LAST = N_NODES - 15 * ROWS

    @functools.partial(
        pl.kernel, mesh=mesh, compiler_params=cp,
        out_type=[
            jax.ShapeDtypeStruct((N_NODES, 128), jnp.float32),
            jax.ShapeDtypeStruct((N_NODES, 128), jnp.float32),
            jax.ShapeDtypeStruct((N_NODES, 128), jnp.float32),
            jax.ShapeDtypeStruct((N_NODES, 128), jnp.float32),
        ],
        scratch_types=[
            pltpu.VMEM_SHARED((N_NODES, 128), jnp.float32),
            pltpu.VMEM((8, 128), jnp.float32),
            pltpu.VMEM((SB, 128), jnp.float32),
            pltpu.VMEM((SB, 128), jnp.float32),
            pltpu.VMEM((SB,), jnp.int32),
            pltpu.VMEM((SB,), jnp.int32),
            pltpu.SemaphoreType.DMA, pltpu.SemaphoreType.DMA,
            pltpu.SemaphoreType.DMA, pltpu.SemaphoreType.DMA,
            pltpu.SemaphoreType.DMA, pltpu.SemaphoreType.DMA,
        ])
    def k(m2a_hbm, m2b_hbm, rlx_hbm, dst_hbm, agga_hbm, aggb_hbm, dpa_hbm,
          dpb_hbm, acc, zb, mb0, mb1, ix0, ix1, si0, sm0, si1, sm1, ss0, ss1):
        c = lax.axis_index("c")
        s = lax.axis_index("s")
        off = s * ROWS
        nz = jnp.where(s == 15, LAST // 8, ROWS // 8)
        zero16 = jnp.zeros((16,), jnp.float32)

        @pl.loop(0, 8)
        def _(i):
            @pl.loop(0, 8)
            def _(j):
                zb[i, pl.ds(j * 16, 16)] = zero16

        def zero_own_rows():
            @pl.loop(0, ROWS // 8)
            def _(i):
                @pl.when(i < nz)
                def _():
                    pltpu.sync_copy(zb, acc.at[pl.ds(off + i * 8, 8)])

        def copy_out(dst_full):
            @pl.when(s < 15)
            def _():
                pltpu.sync_copy(acc.at[pl.ds(off, ROWS)],
                                dst_full.at[pl.ds(off, ROWS)])

            @pl.when(s == 15)
            def _():
                pltpu.sync_copy(acc.at[pl.ds(off, LAST)],
                                dst_full.at[pl.ds(off, LAST)])

        def wait_load(ixb, mub, semi, semm):
            pltpu.make_async_copy(dst_hbm.at[pl.ds(0, SB)], ixb, semi).wait()
            pltpu.make_async_copy(m2a_hbm.at[pl.ds(0, SB)], mub, semm).wait()

        def scat(ixb, mub, sems):
            pltpu.async_copy(mub, acc.at[ixb], sems, add=True)

        def wait_scat(ixb, mub, sems):
            pltpu.make_async_copy(mub, acc.at[ixb], sems).wait()

        def pipelined_phase(nb, load_fn):
            nj = pl.cdiv(nb, 16)

            def valid(j):
                return j * 16 + s < nb

            @pl.when(valid(0))
            def _():
                load_fn(0, ix0, mb0, si0, sm0)

            @pl.loop(0, pl.cdiv(nj, 2))
            def _(it):
                j0 = 2 * it
                j1 = 2 * it + 1
                j2 = 2 * it + 2

                @pl.when(jnp.logical_and(valid(j1), j1 >= 3))
                def _():
                    wait_scat(ix1, mb1, ss1)

                @pl.when(valid(j1))
                def _():
                    load_fn(j1, ix1, mb1, si1, sm1)

                @pl.when(valid(j0))
                def _():
                    wait_load(ix0, mb0, si0, sm0)
                    scat(ix0, mb0, ss0)

                @pl.when(valid(j2))
                def _():
                    wait_scat(ix0, mb0, ss0)
                    load_fn(j2, ix0, mb0, si0, sm0)

                @pl.when(valid(j1))
                def _():
                    wait_load(ix1, mb1, si1, sm1)
                    scat(ix1, mb1, ss1)

            @pl.when(valid(0))
            def _():
                wait_scat(ix0, mb0, ss0)

            @pl.when(valid(1))
            def _():
                wait_scat(ix1, mb1, ss1)

        def load1(j, ixb, mub, semi, semm):
            base = (j * 16 + s) * SB
            pltpu.async_copy(dst_hbm.at[pl.ds(base, SB)], ixb, semi)

            @pl.when(c == 0)
            def _():
                pltpu.async_copy(m2a_hbm.at[pl.ds(base, SB)], mub, semm)

            @pl.when(c == 1)
            def _():
                pltpu.async_copy(m2b_hbm.at[pl.ds(base, SB)], mub, semm)

        zero_own_rows()
        plsc.subcore_barrier()
        pipelined_phase(nblk, load1)
        plsc.subcore_barrier()

        @pl.when(c == 0)
        def _():
            copy_out(agga_hbm)

        @pl.when(c == 1)
        def _():
            copy_out(aggb_hbm)

        def load2(j, ixb, mub, semi, semm):
            base = (c * nblk2 + j * 16 + s) * SB
            pltpu.async_copy(dst_hbm.at[pl.ds(base, SB)], ixb, semi)
            pltpu.async_copy(rlx_hbm.at[pl.ds(base, SB)], mub, semm)

        zero_own_rows()
        plsc.subcore_barrier()
        pipelined_phase(nblk2, load2)
        plsc.subcore_barrier()

        @pl.when(c == 0)
        def _():
            copy_out(dpa_hbm)

        @pl.when(c == 1)
        def _():
            copy_out(dpb_hbm)

    return k(m2a, m2b, rlx128, dst)


def _t3a_body(mask_ref, dpa_ref, dpb_ref, sc_ref, out_ref):
    o = _onehot(mask_ref[...], N_NODES)
    dpos = dpa_ref[:, :16] + dpb_ref[:, :16]
    sums = lax.dot_general(o, dpos, (((0,), (0,)), ((), ())))
    out_ref[...] = sums / sc_ref[:, 0:1]


def _t3a(mask2, dpa, dpb, sc):
    return pl.pallas_call(
        _t3a_body,
        out_shape=jax.ShapeDtypeStruct((NSEG, 16), jnp.float32),
    )(mask2, dpa, dpb, sc)


def _t3b_body(mask_ref, hemb_ref, agga_ref, aggb_ref, dpa_ref, dpb_ref,
              epsc_ref, dmean_ref, sc_ref, wht_ref, whb0_ref, whb1_ref,
              wout_ref, out_ref):
    i = pl.program_id(0)
    o = _onehot(mask_ref[...], NB)
    h_new = _silu(
        jnp.dot(hemb_ref[...].astype(jnp.bfloat16),
                wht_ref[...].astype(jnp.bfloat16),
                preferred_element_type=jnp.float32)
        + jnp.dot(agga_ref[...].astype(jnp.bfloat16),
                  whb0_ref[...].astype(jnp.bfloat16),
                  preferred_element_type=jnp.float32)
        + jnp.dot(aggb_ref[...].astype(jnp.bfloat16),
                  whb1_ref[...].astype(jnp.bfloat16),
                  preferred_element_type=jnp.float32))
    dpos = dpa_ref[:, :16] + dpb_ref[:, :16]
    net16 = (dpos - o @ dmean_ref[...]) + h_new @ wout_ref[...]
    diff = epsc_ref[...] - net16
    err = jnp.sum(diff * diff, axis=1, keepdims=True)
    part = lax.dot_general(o, err, (((0,), (0,)), ((), ())))

    @pl.when(i == 0)
    def _():
        out_ref[...] = part

    @pl.when(i > 0)
    def _():
        out_ref[...] += part

    @pl.when(i == N_NODES // NB - 1)
    def _():
        out_ref[...] *= sc_ref[:, 3:4]


def _t3b(mask2, hemb, agga, aggb, dpa, dpb, epsc, dmean, sc, wht, whb0, whb1,
         wout_pad):
    nblk = N_NODES // NB
    full = lambda r, c: pl.BlockSpec((r, c), lambda i: (0, 0))
    blk = lambda c: pl.BlockSpec((NB, c), lambda i: (i, 0))
    return pl.pallas_call(
        _t3b_body,
        grid=(nblk,),
        in_specs=[
            blk(1), blk(HID), blk(128), blk(128), blk(128), blk(128), blk(16),
            full(NSEG, 16), full(NSEG, 8),
            full(HID, HID), full(128, HID), full(128, HID), full(HID, 16),
        ],
        out_specs=pl.BlockSpec((NSEG, 1), lambda i: (0, 0)),
        out_shape=jax.ShapeDtypeStruct((NSEG, 1), jnp.float32),
    )(mask2, hemb, agga, aggb, dpa, dpb, epsc, dmean, sc, wht, whb0, whb1,
      wout_pad)


def kernel(pos, h, eps, t, conditions, W_in, Wc, We1, We2, Wx, Wh, Wout,
           combined_mask, edge_index):
    f32 = jnp.float32
    mask2 = combined_mask.reshape(N_NODES, 1).astype(jnp.int32)
    src = edge_index[0].astype(jnp.int32)
    dst = edge_index[1].astype(jnp.int32)
    xh = jnp.concatenate([pos, h], axis=1)
    wz = jnp.concatenate([jnp.zeros((3, HID), f32), W_in[:13]], axis=0)
    wt = W_in[13:14]
    perm = jnp.concatenate([jnp.arange(0, HID, 2), jnp.arange(1, HID, 2)])
    we1a = We1[:HID][:, perm]
    we1b = We1[HID:2 * HID][:, perm]
    wd = We1[2 * HID:2 * HID + 1][:, perm]
    We2 = We2[perm, :]
    wht = Wh[:HID]
    whb0 = Wh[HID:HID + 128]
    whb1 = Wh[HID + 128:]
    wout_pad = jnp.concatenate([jnp.zeros((HID, 3), f32), Wout], axis=1)

    mean, sc = _t1a(mask2, eps, t)
    ta, tb, pos4, hemb, epsc = _t1b(mask2, xh, eps, t, conditions, Wc, wz, wt,
                                    we1a, we1b, mean, sc)
    asp, bdp, geo = _s1(ta, tb, pos4.reshape(4 * N_NODES), src, dst)
    m2a, m2b, rlx = _t2(asp, bdp, geo, wd, We2, Wx)
    agga, aggb, dpa, dpb = _s2(m2a, m2b, rlx, dst)
    dmean = _t3a(mask2, dpa, dpb, sc)
    err = _t3b(mask2, hemb, agga, aggb, dpa, dpb, epsc, dmean, sc, wht, whb0,
               whb1, wout_pad)
    return err.reshape(NSEG)


# speedup vs baseline: 5.1123x; 1.0703x over previous
import dataclasses
import functools

import jax
import jax.numpy as jnp
from jax import lax
from jax.experimental import pallas as pl
from jax.experimental.pallas import tpu as pltpu
from jax.experimental.pallas import tpu_sc as plsc

N_NODES = 10000
N_EDGES = 160000
NSEG = 256
HID = 256
TSTEPS = 1000.0

NB = 2000
EB = 1280
GB = 128
SB = 128
TW = 128


def _silu(x):
    return x * lax.logistic(x)


def _pack_pair(x):
    rows = x.shape[0]
    bits = lax.bitcast_convert_type(x, jnp.int32)
    rounded = bits + 0x8000
    lo = lax.shift_right_logical(rounded[:, :TW], 16)
    hi = jnp.bitwise_and(rounded[:, TW:], jnp.int32(-65536))
    return lax.bitcast_convert_type(jnp.bitwise_or(hi, lo), jnp.float32)


def _unpack_pair(p):
    w = lax.bitcast_convert_type(p, jnp.int32)
    lo = lax.bitcast_convert_type(lax.shift_left(w, 16), jnp.float32)
    hi = lax.bitcast_convert_type(
        jnp.bitwise_and(w, jnp.int32(-65536)), jnp.float32)
    return jnp.concatenate([lo, hi], axis=1)


def _onehot(m, rows):
    return (m == lax.broadcasted_iota(jnp.int32, (rows, NSEG), 1)).astype(
        jnp.float32)


def _t1a_body(mask_ref, eps_ref, t_ref, mean_ref, sc_ref):
    o = _onehot(mask_ref[...], N_NODES)
    ones = jnp.ones((N_NODES, 1), jnp.float32)
    cdims = (((0,), (0,)), ((), ()))
    counts = jnp.maximum(lax.dot_general(o, ones, cdims), 1.0)
    sums = lax.dot_general(o, eps_ref[...], cdims)
    mean_ref[...] = sums / counts
    t = t_ref[...]
    gamma_t = -7.0 + 13.0 * t
    gamma_s = -7.0 + 13.0 * (t - 1.0 / TSTEPS)
    alpha = jnp.sqrt(lax.logistic(-gamma_t))
    sigma = jnp.sqrt(lax.logistic(gamma_t))
    snr = 1.0 - jnp.exp(gamma_t - gamma_s)
    sc_ref[...] = jnp.concatenate(
        [counts, alpha, sigma, snr, jnp.zeros((NSEG, 4), jnp.float32)], axis=1)


def _t1a(mask2, eps, t):
    return pl.pallas_call(
        _t1a_body,
        out_shape=[
            jax.ShapeDtypeStruct((NSEG, 16), jnp.float32),
            jax.ShapeDtypeStruct((NSEG, 8), jnp.float32),
        ],
    )(mask2, eps, t)


def _t1b_body(mask_ref, xh_ref, eps_ref, t_ref, cond_ref, wc_ref, wz_ref,
              wt_ref, we1a_ref, we1b_ref, mean_ref, sc_ref,
              ta_ref, tb_ref, pos_ref, hemb_ref, epsc_ref):
    o = _onehot(mask_ref[...], NB)
    sc = sc_ref[...]
    alpha_n = o @ sc[:, 1:2]
    sigma_n = o @ sc[:, 2:3]
    t_n = o @ t_ref[...]
    cmask3 = (lax.broadcasted_iota(jnp.int32, (1, 16), 1) < 3).astype(
        jnp.float32)
    mean_n = (o @ mean_ref[...]) * cmask3
    eps_c = eps_ref[...] - mean_n
    z16 = alpha_n * xh_ref[...] + sigma_n * eps_c
    cond_n = o @ (cond_ref[...] @ wc_ref[...])
    h_emb = _silu(z16 @ wz_ref[...] + t_n @ wt_ref[...] + cond_n)
    cmask4 = (lax.broadcasted_iota(jnp.int32, (1, 4), 1) < 3).astype(
        jnp.float32)
    hb = h_emb.astype(jnp.bfloat16)
    ta32 = jnp.dot(hb, we1a_ref[...].astype(jnp.bfloat16),
                   preferred_element_type=jnp.float32)
    tb32 = jnp.dot(hb, we1b_ref[...].astype(jnp.bfloat16),
                   preferred_element_type=jnp.float32)
    ta_ref[...] = _pack_pair(ta32)
    tb_ref[...] = _pack_pair(tb32)
    pos_ref[...] = z16[:, 0:4] * cmask4
    hemb_ref[...] = h_emb
    epsc_ref[...] = eps_c


def _t1b(mask2, xh, eps, t, conditions, wc, wz, wt, we1a, we1b, mean, sc):
    nblk = N_NODES // NB
    full = lambda r, c: pl.BlockSpec((r, c), lambda i: (0, 0))
    blk = lambda c: pl.BlockSpec((NB, c), lambda i: (i, 0))
    return pl.pallas_call(
        _t1b_body,
        grid=(nblk,),
        in_specs=[
            blk(1), blk(16), blk(16),
            full(NSEG, 1), full(NSEG, 1), full(1, HID),
            full(16, HID), full(1, HID), full(HID, HID), full(HID, HID),
            full(NSEG, 16), full(NSEG, 8),
        ],
        out_specs=[blk(TW), blk(TW), blk(4), blk(HID), blk(16)],
        out_shape=[
            jax.ShapeDtypeStruct((N_NODES, TW), jnp.float32),
            jax.ShapeDtypeStruct((N_NODES, TW), jnp.float32),
            jax.ShapeDtypeStruct((N_NODES, 4), jnp.float32),
            jax.ShapeDtypeStruct((N_NODES, HID), jnp.float32),
            jax.ShapeDtypeStruct((N_NODES, 16), jnp.float32),
        ],
    )(mask2, xh, eps, t, conditions, wc, wz, wt, we1a, we1b, mean, sc)


def _s1(ta, tb, pos4, src, dst):
    mesh = plsc.VectorSubcoreMesh(core_axis_name="c", subcore_axis_name="s")
    cp = pltpu.CompilerParams()
    if "needs_layout_passes" in pltpu.CompilerParams.__dataclass_fields__:
        cp = dataclasses.replace(cp, needs_layout_passes=False)
    nblk = N_EDGES // GB
    nit = pl.cdiv(nblk, 32)

    @functools.partial(
        pl.kernel, mesh=mesh, compiler_params=cp,
        out_type=[
            jax.ShapeDtypeStruct((N_EDGES, TW), jnp.float32),
            jax.ShapeDtypeStruct((N_EDGES, TW), jnp.float32),
            jax.ShapeDtypeStruct((8, N_EDGES), jnp.float32),
        ],
        scratch_types=[
            pltpu.VMEM((GB,), jnp.int32), pltpu.VMEM((GB,), jnp.int32),
            pltpu.VMEM((GB, TW), jnp.float32), pltpu.VMEM((GB, TW), jnp.float32),
            pltpu.VMEM((4 * N_NODES,), jnp.float32),
            pltpu.VMEM((8, GB), jnp.float32),
            pltpu.SemaphoreType.DMA, pltpu.SemaphoreType.DMA,
            pltpu.SemaphoreType.DMA, pltpu.SemaphoreType.DMA,
            pltpu.SemaphoreType.DMA, pltpu.SemaphoreType.DMA,
            pltpu.SemaphoreType.DMA,
        ])
    def k(ta_hbm, tb_hbm, pos_hbm, src_hbm, dst_hbm, asp_hbm, bdp_hbm,
          geo_hbm, si, di, ra, rb, posv, gbuf, sema, semb, sia, sib, swa,
          swb, swg):
        wid = lax.axis_index("s") * 2 + lax.axis_index("c")
        pltpu.sync_copy(pos_hbm, posv)
        zero16 = jnp.zeros((16,), jnp.float32)

        @pl.loop(0, 8)
        def _(g):
            gbuf[3, pl.ds(g * 16, 16)] = zero16
            gbuf[4, pl.ds(g * 16, 16)] = zero16
            gbuf[5, pl.ds(g * 16, 16)] = zero16
            gbuf[6, pl.ds(g * 16, 16)] = zero16
            gbuf[7, pl.ds(g * 16, 16)] = zero16

        @pl.when(wid < nblk)
        def _():
            pltpu.async_copy(src_hbm.at[pl.ds(wid * GB, GB)], si, sia)
            pltpu.async_copy(dst_hbm.at[pl.ds(wid * GB, GB)], di, sib)

        @pl.loop(0, nit)
        def _(it):
            blk = wid + it * 32

            @pl.when(blk < nblk)
            def _():
                base = blk * GB
                pltpu.make_async_copy(src_hbm.at[pl.ds(0, GB)], si, sia).wait()
                pltpu.make_async_copy(dst_hbm.at[pl.ds(0, GB)], di, sib).wait()

                @pl.when(it > 0)
                def _():
                    pltpu.make_async_copy(ra, asp_hbm.at[pl.ds(0, GB)],
                                          swa).wait()
                    pltpu.make_async_copy(rb, bdp_hbm.at[pl.ds(0, GB)],
                                          swb).wait()

                ca = pltpu.async_copy(ta_hbm.at[si], ra, sema)
                cb = pltpu.async_copy(tb_hbm.at[di], rb, semb)

                @pl.when(it > 0)
                def _():
                    pltpu.make_async_copy(gbuf, geo_hbm.at[:, pl.ds(0, GB)],
                                          swg).wait()

                @pl.loop(0, GB // 16)
                def _(g):
                    sidx = si[pl.ds(g * 16, 16)] * 4
                    didx = di[pl.ds(g * 16, 16)] * 4
                    for kc in range(3):
                        ps = plsc.load_gather(posv, [sidx + kc])
                        pd = plsc.load_gather(posv, [didx + kc])
                        gbuf[kc, pl.ds(g * 16, 16)] = ps - pd

                pltpu.async_copy(gbuf, geo_hbm.at[:, pl.ds(base, GB)], swg)
                ca.wait()
                cb.wait()

                @pl.when(blk + 32 < nblk)
                def _():
                    pltpu.async_copy(src_hbm.at[pl.ds((blk + 32) * GB, GB)],
                                     si, sia)
                    pltpu.async_copy(dst_hbm.at[pl.ds((blk + 32) * GB, GB)],
                                     di, sib)

                pltpu.async_copy(ra, asp_hbm.at[pl.ds(base, GB)], swa)
                pltpu.async_copy(rb, bdp_hbm.at[pl.ds(base, GB)], swb)

        pltpu.make_async_copy(ra, asp_hbm.at[pl.ds(0, GB)], swa).wait()
        pltpu.make_async_copy(rb, bdp_hbm.at[pl.ds(0, GB)], swb).wait()
        pltpu.make_async_copy(gbuf, geo_hbm.at[:, pl.ds(0, GB)], swg).wait()

    return k(ta, tb, pos4, src, dst)


def _t2_body(a_ref, b_ref, g_ref, wd_ref, we2_ref, wx_ref, m2a_ref, m2b_ref,
             rlx_ref):
    a = _unpack_pair(a_ref[...])
    b = _unpack_pair(b_ref[...])
    eye8 = (lax.broadcasted_iota(jnp.int32, (8, 8), 0)
            == lax.broadcasted_iota(jnp.int32, (8, 8), 1)).astype(jnp.float32)
    rel = lax.dot_general(g_ref[...], eye8, (((0,), (0,)), ((), ())))
    d2 = jnp.sum(rel * rel, axis=1, keepdims=True)
    m1 = _silu(a + b + d2 @ wd_ref[...])
    m2 = _silu(jnp.dot(m1.astype(jnp.bfloat16),
                       we2_ref[...].astype(jnp.bfloat16),
                       preferred_element_type=jnp.float32))
    wx = m2 @ wx_ref[...]
    m2a_ref[...] = m2[:, :128]
    m2b_ref[...] = m2[:, 128:]
    pad128 = (lax.broadcasted_iota(jnp.int32, (8, 128), 0)
              == lax.broadcasted_iota(jnp.int32, (8, 128), 1)).astype(
                  jnp.float32)
    rlx_ref[...] = (rel * wx) @ pad128


def _t2(asp, bdp, geo, wd, we2, wx):
    nblk = N_EDGES // EB
    full = lambda r, c: pl.BlockSpec((r, c), lambda i: (0, 0))
    blk = lambda c: pl.BlockSpec((EB, c), lambda i: (i, 0))
    return pl.pallas_call(
        _t2_body,
        grid=(nblk,),
        in_specs=[blk(TW), blk(TW), pl.BlockSpec((8, EB), lambda i: (0, i)),
                  full(1, HID), full(HID, HID), full(HID, 1)],
        out_specs=[blk(128), blk(128), blk(128)],
        out_shape=[
            jax.ShapeDtypeStruct((N_EDGES, 128), jnp.float32),
            jax.ShapeDtypeStruct((N_EDGES, 128), jnp.float32),
            jax.ShapeDtypeStruct((N_EDGES, 128), jnp.float32),
        ],
    )(asp, bdp, geo, wd, we2, wx)


def _s2(m2a, m2b, rlx128, dst):
    mesh = plsc.VectorSubcoreMesh(core_axis_name="c", subcore_axis_name="s")
    cp = pltpu.CompilerParams()
    if "needs_layout_passes" in pltpu.CompilerParams.__dataclass_fields__:
        cp = dataclasses.replace(cp, needs_layout_passes=False)
    nblk = N_EDGES // SB
    nblk2 = nblk // 2
    ROWS = 632
    LAST = N_NODES - 15 * ROWS

    @functools.partial(
        pl.kernel, mesh=mesh, compiler_params=cp,
        out_type=[
            jax.ShapeDtypeStruct((N_NODES, 128), jnp.float32),
            jax.ShapeDtypeStruct((N_NODES, 128), jnp.float32),
            jax.ShapeDtypeStruct((N_NODES, 128), jnp.float32),
            jax.ShapeDtypeStruct((N_NODES, 128), jnp.float32),
        ],
        scratch_types=[
            pltpu.VMEM_SHARED((N_NODES, 128), jnp.float32),
            pltpu.VMEM((8, 128), jnp.float32),
            pltpu.VMEM((SB, 128), jnp.float32),
            pltpu.VMEM((SB, 128), jnp.float32),
            pltpu.VMEM((SB,), jnp.int32),
            pltpu.VMEM((SB,), jnp.int32),
            pltpu.SemaphoreType.DMA, pltpu.SemaphoreType.DMA,
            pltpu.SemaphoreType.DMA, pltpu.SemaphoreType.DMA,
            pltpu.SemaphoreType.DMA, pltpu.SemaphoreType.DMA,
        ])
    def k(m2a_hbm, m2b_hbm, rlx_hbm, dst_hbm, agga_hbm, aggb_hbm, dpa_hbm,
          dpb_hbm, acc, zb, mb0, mb1, ix0, ix1, si0, sm0, si1, sm1, ss0, ss1):
        c = lax.axis_index("c")
        s = lax.axis_index("s")
        off = s * ROWS
        nz = jnp.where(s == 15, LAST // 8, ROWS // 8)
        zero16 = jnp.zeros((16,), jnp.float32)

        @pl.loop(0, 8)
        def _(i):
            @pl.loop(0, 8)
            def _(j):
                zb[i, pl.ds(j * 16, 16)] = zero16

        def zero_own_rows():
            @pl.loop(0, ROWS // 8)
            def _(i):
                @pl.when(i < nz)
                def _():
                    pltpu.sync_copy(zb, acc.at[pl.ds(off + i * 8, 8)])

        def copy_out(dst_full):
            @pl.when(s < 15)
            def _():
                pltpu.sync_copy(acc.at[pl.ds(off, ROWS)],
                                dst_full.at[pl.ds(off, ROWS)])

            @pl.when(s == 15)
            def _():
                pltpu.sync_copy(acc.at[pl.ds(off, LAST)],
                                dst_full.at[pl.ds(off, LAST)])

        def wait_load(ixb, mub, semi, semm):
            pltpu.make_async_copy(dst_hbm.at[pl.ds(0, SB)], ixb, semi).wait()
            pltpu.make_async_copy(m2a_hbm.at[pl.ds(0, SB)], mub, semm).wait()

        def scat(ixb, mub, sems):
            pltpu.async_copy(mub, acc.at[ixb], sems, add=True)

        def wait_scat(ixb, mub, sems):
            pltpu.make_async_copy(mub, acc.at[ixb], sems).wait()

        def pipelined_phase(nb, load_fn):
            nj = pl.cdiv(nb, 16)

            def valid(j):
                return j * 16 + s < nb

            @pl.when(valid(0))
            def _():
                load_fn(0, ix0, mb0, si0, sm0)

            @pl.loop(0, pl.cdiv(nj, 2))
            def _(it):
                j0 = 2 * it
                j1 = 2 * it + 1
                j2 = 2 * it + 2

                @pl.when(jnp.logical_and(valid(j1), j1 >= 3))
                def _():
                    wait_scat(ix1, mb1, ss1)

                @pl.when(valid(j1))
                def _():
                    load_fn(j1, ix1, mb1, si1, sm1)

                @pl.when(valid(j0))
                def _():
                    wait_load(ix0, mb0, si0, sm0)
                    scat(ix0, mb0, ss0)

                @pl.when(valid(j2))
                def _():
                    wait_scat(ix0, mb0, ss0)
                    load_fn(j2, ix0, mb0, si0, sm0)

                @pl.when(valid(j1))
                def _():
                    wait_load(ix1, mb1, si1, sm1)
                    scat(ix1, mb1, ss1)

            @pl.when(valid(0))
            def _():
                wait_scat(ix0, mb0, ss0)

            @pl.when(valid(1))
            def _():
                wait_scat(ix1, mb1, ss1)

        def load1(j, ixb, mub, semi, semm):
            base = (j * 16 + s) * SB
            pltpu.async_copy(dst_hbm.at[pl.ds(base, SB)], ixb, semi)

            @pl.when(c == 0)
            def _():
                pltpu.async_copy(m2a_hbm.at[pl.ds(base, SB)], mub, semm)

            @pl.when(c == 1)
            def _():
                pltpu.async_copy(m2b_hbm.at[pl.ds(base, SB)], mub, semm)

        zero_own_rows()
        plsc.subcore_barrier()
        pipelined_phase(nblk, load1)
        plsc.subcore_barrier()

        @pl.when(c == 0)
        def _():
            copy_out(agga_hbm)

        @pl.when(c == 1)
        def _():
            copy_out(aggb_hbm)

        def load2(j, ixb, mub, semi, semm):
            base = (c * nblk2 + j * 16 + s) * SB
            pltpu.async_copy(dst_hbm.at[pl.ds(base, SB)], ixb, semi)
            pltpu.async_copy(rlx_hbm.at[pl.ds(base, SB)], mub, semm)

        zero_own_rows()
        plsc.subcore_barrier()
        pipelined_phase(nblk2, load2)
        plsc.subcore_barrier()

        @pl.when(c == 0)
        def _():
            copy_out(dpa_hbm)

        @pl.when(c == 1)
        def _():
            copy_out(dpb_hbm)

    return k(m2a, m2b, rlx128, dst)


def _t3a_body(mask_ref, dpa_ref, dpb_ref, sc_ref, out_ref):
    o = _onehot(mask_ref[...], N_NODES)
    dpos = dpa_ref[:, :16] + dpb_ref[:, :16]
    sums = lax.dot_general(o, dpos, (((0,), (0,)), ((), ())))
    out_ref[...] = sums / sc_ref[:, 0:1]


def _t3a(mask2, dpa, dpb, sc):
    return pl.pallas_call(
        _t3a_body,
        out_shape=jax.ShapeDtypeStruct((NSEG, 16), jnp.float32),
    )(mask2, dpa, dpb, sc)


def _t3b_body(mask_ref, hemb_ref, agga_ref, aggb_ref, dpa_ref, dpb_ref,
              epsc_ref, dmean_ref, sc_ref, wht_ref, whb0_ref, whb1_ref,
              wout_ref, out_ref):
    i = pl.program_id(0)
    o = _onehot(mask_ref[...], NB)
    h_new = _silu(
        jnp.dot(hemb_ref[...].astype(jnp.bfloat16),
                wht_ref[...].astype(jnp.bfloat16),
                preferred_element_type=jnp.float32)
        + jnp.dot(agga_ref[...].astype(jnp.bfloat16),
                  whb0_ref[...].astype(jnp.bfloat16),
                  preferred_element_type=jnp.float32)
        + jnp.dot(aggb_ref[...].astype(jnp.bfloat16),
                  whb1_ref[...].astype(jnp.bfloat16),
                  preferred_element_type=jnp.float32))
    dpos = dpa_ref[:, :16] + dpb_ref[:, :16]
    net16 = (dpos - o @ dmean_ref[...]) + h_new @ wout_ref[...]
    diff = epsc_ref[...] - net16
    err = jnp.sum(diff * diff, axis=1, keepdims=True)
    part = lax.dot_general(o, err, (((0,), (0,)), ((), ())))

    @pl.when(i == 0)
    def _():
        out_ref[...] = part

    @pl.when(i > 0)
    def _():
        out_ref[...] += part

    @pl.when(i == N_NODES // NB - 1)
    def _():
        out_ref[...] *= sc_ref[:, 3:4]


def _t3b(mask2, hemb, agga, aggb, dpa, dpb, epsc, dmean, sc, wht, whb0, whb1,
         wout_pad):
    nblk = N_NODES // NB
    full = lambda r, c: pl.BlockSpec((r, c), lambda i: (0, 0))
    blk = lambda c: pl.BlockSpec((NB, c), lambda i: (i, 0))
    return pl.pallas_call(
        _t3b_body,
        grid=(nblk,),
        in_specs=[
            blk(1), blk(HID), blk(128), blk(128), blk(128), blk(128), blk(16),
            full(NSEG, 16), full(NSEG, 8),
            full(HID, HID), full(128, HID), full(128, HID), full(HID, 16),
        ],
        out_specs=pl.BlockSpec((NSEG, 1), lambda i: (0, 0)),
        out_shape=jax.ShapeDtypeStruct((NSEG, 1), jnp.float32),
    )(mask2, hemb, agga, aggb, dpa, dpb, epsc, dmean, sc, wht, whb0, whb1,
      wout_pad)


def kernel(pos, h, eps, t, conditions, W_in, Wc, We1, We2, Wx, Wh, Wout,
           combined_mask, edge_index):
    f32 = jnp.float32
    mask2 = combined_mask.reshape(N_NODES, 1).astype(jnp.int32)
    src = edge_index[0].astype(jnp.int32)
    dst = edge_index[1].astype(jnp.int32)
    xh = jnp.concatenate([pos, h], axis=1)
    wz = jnp.concatenate([jnp.zeros((3, HID), f32), W_in[:13]], axis=0)
    wt = W_in[13:14]
    perm = jnp.concatenate([jnp.arange(0, HID, 2), jnp.arange(1, HID, 2)])
    we1a = We1[:HID][:, perm]
    we1b = We1[HID:2 * HID][:, perm]
    wd = We1[2 * HID:2 * HID + 1][:, perm]
    We2 = We2[perm, :]
    wht = Wh[:HID]
    whb0 = Wh[HID:HID + 128]
    whb1 = Wh[HID + 128:]
    wout_pad = jnp.concatenate([jnp.zeros((HID, 3), f32), Wout], axis=1)

    mean, sc = _t1a(mask2, eps, t)
    ta, tb, pos4, hemb, epsc = _t1b(mask2, xh, eps, t, conditions, Wc, wz, wt,
                                    we1a, we1b, mean, sc)
    asp, bdp, geo = _s1(ta, tb, pos4.reshape(4 * N_NODES), src, dst)
    m2a, m2b, rlx = _t2(asp, bdp, geo, wd, We2, Wx)
    agga, aggb, dpa, dpb = _s2(m2a, m2b, rlx, dst)
    dmean = _t3a(mask2, dpa, dpb, sc)
    err = _t3b(mask2, hemb, agga, aggb, dpa, dpb, epsc, dmean, sc, wht, whb0,
               whb1, wout_pad)
    return err.reshape(NSEG)
